# Initial kernel scaffold; baseline (speedup 1.0000x reference)
import jax
import jax.numpy as jnp
from jax.experimental import pallas as pl


def kernel(edge_index_m, edge_index_d, data_m, data_d, x_m, x_d, beta1, beta2, gcn_x1_W, gcn_x1_b, gcn_x2_W, gcn_x2_b, gcn_x3_W, gcn_x3_b, gcn_y1_W, gcn_y1_b, gcn_y2_W, gcn_y2_b, gcn_y3_W, gcn_y3_b, gat_x_W, gat_x_as, gat_x_ad, gat_x_b, gat_y_W, gat_y_as, gat_y_ad, gat_y_b, lin_x1_W, lin_x1_b, lin_x2_W, lin_x2_b, lin_x3_W, lin_x3_b, lin_y1_W, lin_y1_b, lin_y2_W, lin_y2_b, lin_y3_W, lin_y3_b):
    raise NotImplementedError("write your pallas kernel here")


# speedup vs baseline: 6.0130x; 6.0130x over previous
import functools

import jax
import jax.numpy as jnp
from jax import lax
from jax.experimental import pallas as pl
from jax.experimental.pallas import tpu as pltpu
from jax.experimental.pallas import tpu_sc as plsc

NN = 5000
NPAD = 5120
NE = 160000
EP = NE + NPAD
NT = 16
CH = 80
K1C = (NE // NT) // CH
RPT = NPAD // NT
F = 256
NHEAD = 4
FZ = F + 16
CAP = 16000
CAPC = CAP // CH

_MESH = dict(core_axis_name="c", subcore_axis_name="s")


def _lr(v):
    return jnp.where(v > 0, v, v * 0.01)


def _qrsqrt(x):
    i = lax.bitcast_convert_type(x, jnp.int32)
    i = jnp.int32(0x5F3759DF) - lax.shift_right_logical(i, 1)
    y = lax.bitcast_convert_type(i, jnp.float32)
    for _ in range(3):
        y = y * (1.5 - 0.5 * x * y * y)
    return y


def _sc_prep(data_m_flat, data_d_flat, flat_k1, src_k1, dst_k1):
    mesh = plsc.VectorSubcoreMesh(**_MESH)

    @functools.partial(
        pl.kernel,
        out_type=jax.ShapeDtypeStruct((2, 1, EP), jnp.float32),
        mesh=mesh,
        compiler_params=pltpu.CompilerParams(needs_layout_passes=False),
        scratch_types=[
            pltpu.VMEM((K1C, CH), jnp.int32),
            pltpu.VMEM((K1C, CH), jnp.int32),
            pltpu.VMEM((K1C, CH), jnp.float32),
            pltpu.VMEM((RPT,), jnp.float32),
            pltpu.VMEM((NPAD,), jnp.float32),
            pltpu.VMEM_SHARED((NPAD,), jnp.float32),
            pltpu.SemaphoreType.DMA,
        ],
    )
    def k(dm_ref, dd_ref, flat_ref, src_ref, dst_ref, coef_ref,
          fbuf, dbuf, wbuf, dv, dinvtab, deg_sh, sem):
        c = lax.axis_index("c")
        s = lax.axis_index("s")
        pltpu.sync_copy(flat_ref.at[c, s], fbuf)
        pltpu.sync_copy(dst_ref.at[c, s], dbuf)
        for j in range(RPT // 16):
            dv[pl.ds(j * 16, 16)] = jnp.full((16,), 1.0, jnp.float32)
        pltpu.sync_copy(dv, deg_sh.at[pl.ds(s * RPT, RPT)])
        plsc.subcore_barrier()

        def gather_all(dref):
            def chunk(kk, _):
                pltpu.async_copy(dref.at[fbuf.at[kk]], wbuf.at[kk], sem).wait()
                return 0
            lax.fori_loop(0, K1C, chunk, 0)

        @pl.when(c == 0)
        def _():
            gather_all(dm_ref)

        @pl.when(c == 1)
        def _():
            gather_all(dd_ref)

        def degadd(kk, _):
            pltpu.sync_copy(wbuf.at[kk], deg_sh.at[dbuf.at[kk]], add=True)
            return 0
        lax.fori_loop(0, K1C, degadd, 0)
        plsc.subcore_barrier()

        pltpu.sync_copy(deg_sh.at[pl.ds(s * RPT, RPT)], dv)
        for j in range(RPT // 16):
            dv[pl.ds(j * 16, 16)] = _qrsqrt(dv[pl.ds(j * 16, 16)])
        pltpu.sync_copy(dv, deg_sh.at[pl.ds(s * RPT, RPT)])
        for j in range(RPT // 16):
            y = dv[pl.ds(j * 16, 16)]
            dv[pl.ds(j * 16, 16)] = y * y
        pltpu.sync_copy(dv, coef_ref.at[c, 0].at[pl.ds(NE + s * RPT, RPT)])
        plsc.subcore_barrier()

        pltpu.sync_copy(deg_sh, dinvtab)
        pltpu.sync_copy(src_ref.at[c, s], fbuf)

        def normchunk(kk, _):
            def sub(i, _):
                sl = pl.ds(i * 16, 16)
                sv = fbuf[kk, sl]
                tv = dbuf[kk, sl]
                dsv = plsc.load_gather(dinvtab, [sv])
                dtv = plsc.load_gather(dinvtab, [tv])
                wbuf[kk, sl] = dsv * wbuf[kk, sl] * dtv
                return 0
            lax.fori_loop(0, CH // 16, sub, 0)
            pltpu.sync_copy(
                wbuf.at[kk],
                coef_ref.at[c, 0].at[pl.ds(s * (K1C * CH) + kk * CH, CH)])
            return 0
        lax.fori_loop(0, K1C, normchunk, 0)

    return k(data_m_flat, data_d_flat, flat_k1, src_k1, dst_k1)


def _sc_permute(src_all, dst_all, coeff, permp, seglen):
    mesh = plsc.VectorSubcoreMesh(**_MESH)

    @functools.partial(
        pl.kernel,
        out_type=(
            jax.ShapeDtypeStruct((2, NT, 1, CAP), jnp.int32),
            jax.ShapeDtypeStruct((2, NT, 1, CAP), jnp.int32),
            jax.ShapeDtypeStruct((2, NT, 1, CAP), jnp.float32),
            jax.ShapeDtypeStruct((2, NT, 1, CAP), jnp.float32),
        ),
        mesh=mesh,
        compiler_params=pltpu.CompilerParams(needs_layout_passes=False),
        scratch_types=[
            pltpu.VMEM((CH,), jnp.int32),
            pltpu.VMEM((CH,), jnp.int32),
            pltpu.VMEM((CH,), jnp.float32),
            pltpu.VMEM((1, 16), jnp.int32),
            pltpu.SemaphoreType.DMA,
        ],
    )
    def k(src_ref, dst_ref, coef_ref, perm_ref, len_ref,
          psrc_ref, pdst_ref, pcoef_ref, pval_ref, pbuf, ibuf, cbuf, lbuf,
          sem):
        c = lax.axis_index("c")
        s = lax.axis_index("s")
        pltpu.sync_copy(len_ref.at[c], lbuf)
        seg = plsc.load_gather(
            lbuf, [jnp.zeros((16,), jnp.int32),
                   jnp.full((16,), s, jnp.int32)])[0]
        iota = lax.iota(jnp.int32, 16)

        def chunk(kk, _):
            pltpu.sync_copy(perm_ref.at[c, s, 0].at[pl.ds(kk * CH, CH)], pbuf)
            base = kk * CH
            pltpu.async_copy(src_ref.at[pbuf], ibuf, sem).wait()
            for i in range(CH // 16):
                sl = pl.ds(i * 16, 16)
                ibuf[sl] = ibuf[sl] + c * NPAD
            pltpu.sync_copy(ibuf, psrc_ref.at[c, s, 0].at[pl.ds(base, CH)])
            pltpu.async_copy(dst_ref.at[pbuf], ibuf, sem).wait()
            for i in range(CH // 16):
                sl = pl.ds(i * 16, 16)
                valid = (base + i * 16 + iota) < seg
                ibuf[sl] = jnp.where(valid, ibuf[sl] - s * RPT, 0)
            pltpu.sync_copy(ibuf, pdst_ref.at[c, s, 0].at[pl.ds(base, CH)])
            pltpu.async_copy(coef_ref.at[pbuf], cbuf, sem).wait()
            for i in range(CH // 16):
                sl = pl.ds(i * 16, 16)
                valid = (base + i * 16 + iota) < seg
                cbuf[sl] = jnp.where(valid, cbuf[sl], 0.0)
            pltpu.sync_copy(cbuf, pcoef_ref.at[c, s, 0].at[pl.ds(base, CH)])
            for i in range(CH // 16):
                sl = pl.ds(i * 16, 16)
                valid = (base + i * 16 + iota) < seg
                cbuf[sl] = jnp.where(valid, 1.0, 0.0)
            pltpu.sync_copy(cbuf, pval_ref.at[c, s, 0].at[pl.ds(base, CH)])
            return 0
        nch = lax.div(seg + (CH - 1), CH)
        lax.fori_loop(0, nch, chunk, 0)
        for i in range(CH // 16):
            ibuf[pl.ds(i * 16, 16)] = jnp.zeros((16,), jnp.int32)
            cbuf[pl.ds(i * 16, 16)] = jnp.zeros((16,), jnp.float32)

        def padchunk(kk, _):
            base = kk * CH
            pltpu.sync_copy(ibuf, psrc_ref.at[c, s, 0].at[pl.ds(base, CH)])
            pltpu.sync_copy(ibuf, pdst_ref.at[c, s, 0].at[pl.ds(base, CH)])
            pltpu.sync_copy(cbuf, pcoef_ref.at[c, s, 0].at[pl.ds(base, CH)])
            pltpu.sync_copy(cbuf, pval_ref.at[c, s, 0].at[pl.ds(base, CH)])
            return 0
        lax.fori_loop(nch, CAPC, padchunk, 0)

    return k(src_all, dst_all, coeff, permp, seglen)


def _sc_prop(feat2d, psrc, pdst, pcoef, nchunks):
    mesh = plsc.VectorSubcoreMesh(**_MESH)

    @functools.partial(
        pl.kernel,
        out_type=jax.ShapeDtypeStruct((2, NPAD, F), jnp.float32),
        mesh=mesh,
        compiler_params=pltpu.CompilerParams(needs_layout_passes=False),
        scratch_types=[
            pltpu.VMEM((CH,), jnp.int32),
            pltpu.VMEM((CH,), jnp.int32),
            pltpu.VMEM((CH,), jnp.float32),
            pltpu.VMEM((CH, F), jnp.float32),
            pltpu.VMEM((RPT, F), jnp.float32),
            pltpu.VMEM((1, 16), jnp.int32),
            pltpu.SemaphoreType.DMA,
        ],
    )
    def k(feat_ref, src_ref, dst_ref, coef_ref, nch_ref, out_ref,
          sbuf, dbuf, cbuf, rows, acc, lbuf, sem):
        c = lax.axis_index("c")
        s = lax.axis_index("s")
        pltpu.sync_copy(nch_ref.at[c], lbuf)
        nch = plsc.load_gather(
            lbuf, [jnp.zeros((16,), jnp.int32),
                   jnp.full((16,), s, jnp.int32)])[0]

        def zr(r, _):
            for j in range(F // 16):
                acc[r, pl.ds(j * 16, 16)] = jnp.zeros((16,), jnp.float32)
            return 0
        lax.fori_loop(0, RPT, zr, 0)

        def chunk(kk, _):
            base = kk * CH
            pltpu.sync_copy(src_ref.at[c, s, 0].at[pl.ds(base, CH)], sbuf)
            pltpu.sync_copy(dst_ref.at[c, s, 0].at[pl.ds(base, CH)], dbuf)
            pltpu.sync_copy(coef_ref.at[c, s, 0].at[pl.ds(base, CH)], cbuf)
            pltpu.async_copy(feat_ref.at[sbuf], rows, sem).wait()

            def grp(i, _):
                sl = pl.ds(i * 16, 16)
                cvec = cbuf[sl]
                dvec = dbuf[sl]
                for e in range(16):
                    cc = cvec[e]
                    dl = dvec[e]
                    r = i * 16 + e
                    for j in range(F // 16):
                        fs = pl.ds(j * 16, 16)
                        acc[dl, fs] = acc[dl, fs] + rows[r, fs] * cc
                return 0
            lax.fori_loop(0, CH // 16, grp, 0)
            return 0
        lax.fori_loop(0, nch, chunk, 0)
        pltpu.sync_copy(acc, out_ref.at[c, pl.ds(s * RPT, RPT), :])

    return k(feat2d, psrc, pdst, pcoef, nchunks)


def _sc_gat(featx, adfull, psrc, pdst, pval, nchunks):
    mesh = plsc.VectorSubcoreMesh(**_MESH)
    CHG = 64

    @functools.partial(
        pl.kernel,
        out_type=(jax.ShapeDtypeStruct((2, NPAD, F), jnp.float32),
                  jax.ShapeDtypeStruct((2, NT, 1, RPT * 16), jnp.float32)),
        mesh=mesh,
        compiler_params=pltpu.CompilerParams(needs_layout_passes=False),
        scratch_types=[
            pltpu.VMEM((CHG,), jnp.int32),
            pltpu.VMEM((CHG,), jnp.int32),
            pltpu.VMEM((CHG,), jnp.float32),
            pltpu.VMEM((CHG,), jnp.int32),
            pltpu.VMEM((CHG, F + 128), jnp.float32),
            pltpu.VMEM((CHG, 128), jnp.float32),
            pltpu.VMEM((RPT, F), jnp.float32),
            pltpu.VMEM((RPT * 16,), jnp.float32),
            pltpu.VMEM((1, 16), jnp.int32),
            pltpu.SemaphoreType.DMA,
            pltpu.SemaphoreType.DMA,
        ],
    )
    def k(feat_ref, ad_ref, src_ref, dst_ref, val_ref, nch_ref,
          out_ref, z_ref,
          sbuf, dbuf, vbuf, gbuf, rows, adrows, acc, accz, lbuf, sem, sem2):
        c = lax.axis_index("c")
        s = lax.axis_index("s")
        pltpu.sync_copy(nch_ref.at[c], lbuf)
        nch = plsc.load_gather(
            lbuf, [jnp.zeros((16,), jnp.int32),
                   jnp.full((16,), s, jnp.int32)])[0]

        def zr(r, _):
            for j in range(F // 16):
                acc[r, pl.ds(j * 16, 16)] = jnp.zeros((16,), jnp.float32)
            accz[pl.ds(r * 16, 16)] = jnp.zeros((16,), jnp.float32)
            return 0
        lax.fori_loop(0, RPT, zr, 0)

        goff = c * NPAD + s * RPT

        def chunk(kk, _):
            base = kk * CHG
            pltpu.sync_copy(src_ref.at[c, s, 0].at[pl.ds(base, CHG)], sbuf)
            pltpu.sync_copy(dst_ref.at[c, s, 0].at[pl.ds(base, CHG)], dbuf)
            pltpu.sync_copy(val_ref.at[c, s, 0].at[pl.ds(base, CHG)], vbuf)
            for i in range(CHG // 16):
                sl = pl.ds(i * 16, 16)
                gbuf[sl] = dbuf[sl] + goff
            cp1 = pltpu.async_copy(feat_ref.at[sbuf], rows, sem)
            cp2 = pltpu.async_copy(ad_ref.at[gbuf], adrows, sem2)
            cp1.wait()
            cp2.wait()

            def grp(i, _):
                sl = pl.ds(i * 16, 16)
                dvec = dbuf[sl]
                vvec = vbuf[sl]
                for e in range(16):
                    dl = dvec[e]
                    r = i * 16 + e
                    x = rows[r, pl.ds(F, 16)] + adrows[r, pl.ds(0, 16)]
                    x = jnp.maximum(x, x * 0.2)
                    pv = jnp.exp(x) * vvec[e]
                    for h in range(NHEAD):
                        ph = pv[h]
                        for j in range(F // (16 * NHEAD)):
                            fs = pl.ds(h * (F // NHEAD) + j * 16, 16)
                            acc[dl, fs] = acc[dl, fs] + rows[r, fs] * ph
                    accz[pl.ds(dl * 16, 16)] = accz[pl.ds(dl * 16, 16)] + pv
                return 0
            lax.fori_loop(0, CHG // 16, grp, 0)
            return 0
        lax.fori_loop(0, nch, chunk, 0)
        pltpu.sync_copy(acc, out_ref.at[c, pl.ds(s * RPT, RPT), :])
        pltpu.sync_copy(accz, z_ref.at[c, s, 0])

    return k(featx, adfull, psrc, pdst, pval, nchunks)


BM = 512
GB = NPAD // BM


def _tc_mm(x, w):
    ki, ko = w.shape[1], w.shape[2]

    def body(x_ref, w_ref, o_ref):
        o_ref[...] = jnp.dot(x_ref[0], w_ref[0],
                             preferred_element_type=jnp.float32)[None]

    return pl.pallas_call(
        body,
        grid=(2, GB),
        in_specs=[pl.BlockSpec((1, BM, ki), lambda g, i: (g, i, 0)),
                  pl.BlockSpec((1, ki, ko), lambda g, i: (g, 0, 0))],
        out_specs=pl.BlockSpec((1, BM, ko), lambda g, i: (g, i, 0)),
        out_shape=jax.ShapeDtypeStruct((2, NPAD, ko), jnp.float32),
    )(x, w)


def _tc_lrelu_mm(acc, b, w):
    ko = w.shape[2]

    def body(a_ref, b_ref, w_ref, x_ref, xw_ref):
        X = _lr(a_ref[0] + b_ref[0])
        x_ref[...] = X[None]
        xw_ref[...] = jnp.dot(X, w_ref[0],
                              preferred_element_type=jnp.float32)[None]

    return pl.pallas_call(
        body,
        grid=(2, GB),
        in_specs=[pl.BlockSpec((1, BM, F), lambda g, i: (g, i, 0)),
                  pl.BlockSpec((1, 1, F), lambda g, i: (g, 0, 0)),
                  pl.BlockSpec((1, F, ko), lambda g, i: (g, 0, 0))],
        out_specs=[pl.BlockSpec((1, BM, F), lambda g, i: (g, i, 0)),
                   pl.BlockSpec((1, BM, ko), lambda g, i: (g, i, 0))],
        out_shape=[jax.ShapeDtypeStruct((2, NPAD, F), jnp.float32),
                   jax.ShapeDtypeStruct((2, NPAD, ko), jnp.float32)],
    )(acc, b, w)


def _tc_gat_post(acc, zz, bg, w2, rmat):
    def body(a_ref, z_ref, bg_ref, w_ref, r_ref, x_ref, xw_ref):
        zr = jnp.dot(z_ref[0], r_ref[...], preferred_element_type=jnp.float32)
        X = _lr(a_ref[0] / jnp.maximum(zr, 1e-16) + bg_ref[0])
        x_ref[...] = X[None]
        xw_ref[...] = jnp.dot(X, w_ref[0],
                              preferred_element_type=jnp.float32)[None]

    return pl.pallas_call(
        body,
        grid=(2, GB),
        in_specs=[pl.BlockSpec((1, BM, F), lambda g, i: (g, i, 0)),
                  pl.BlockSpec((1, BM, 16), lambda g, i: (g, i, 0)),
                  pl.BlockSpec((1, 1, F), lambda g, i: (g, 0, 0)),
                  pl.BlockSpec((1, F, F), lambda g, i: (g, 0, 0)),
                  pl.BlockSpec((16, F), lambda g, i: (0, 0))],
        out_specs=[pl.BlockSpec((1, BM, F), lambda g, i: (g, i, 0)),
                   pl.BlockSpec((1, BM, F), lambda g, i: (g, i, 0))],
        out_shape=[jax.ShapeDtypeStruct((2, NPAD, F), jnp.float32),
                   jax.ShapeDtypeStruct((2, NPAD, F), jnp.float32)],
    )(acc, zz, bg, w2, rmat)


def _tc_mlp(acc3, b3, x1g, x2, betas, l1, c1, l2, c2, l3, c3):
    def body(a_ref, b_ref, x1_ref, x2_ref, bt_ref,
             l1_ref, c1_ref, l2_ref, c2_ref, l3_ref, c3_ref, o_ref):
        g = pl.program_id(0)
        b0 = bt_ref[g, 0]
        b1 = bt_ref[g, 1]
        X3 = _lr(a_ref[0] + b_ref[0])
        X = b0 * x1_ref[0] + b1 * x2_ref[0] + (1.0 - b0 - b1) * X3
        h = _lr(jnp.dot(X, l1_ref[0], preferred_element_type=jnp.float32)
                + c1_ref[0])
        h = _lr(jnp.dot(h, l2_ref[0], preferred_element_type=jnp.float32)
                + c2_ref[0])
        h = _lr(jnp.dot(h, l3_ref[0], preferred_element_type=jnp.float32)
                + c3_ref[0])
        o_ref[...] = h[None]

    return pl.pallas_call(
        body,
        grid=(2, GB),
        in_specs=[pl.BlockSpec((1, BM, F), lambda g, i: (g, i, 0)),
                  pl.BlockSpec((1, 1, F), lambda g, i: (g, 0, 0)),
                  pl.BlockSpec((1, BM, F), lambda g, i: (g, i, 0)),
                  pl.BlockSpec((1, BM, F), lambda g, i: (g, i, 0)),
                  pl.BlockSpec(memory_space=pltpu.SMEM),
                  pl.BlockSpec((1, F, F), lambda g, i: (g, 0, 0)),
                  pl.BlockSpec((1, 1, F), lambda g, i: (g, 0, 0)),
                  pl.BlockSpec((1, F, 128), lambda g, i: (g, 0, 0)),
                  pl.BlockSpec((1, 1, 128), lambda g, i: (g, 0, 0)),
                  pl.BlockSpec((1, 128, 64), lambda g, i: (g, 0, 0)),
                  pl.BlockSpec((1, 1, 64), lambda g, i: (g, 0, 0))],
        out_specs=pl.BlockSpec((1, BM, 64), lambda g, i: (g, i, 0)),
        out_shape=jax.ShapeDtypeStruct((2, NPAD, 64), jnp.float32),
    )(acc3, b3, x1g, x2, betas, l1, c1, l2, c2, l3, c3)


def _tc_final(feats):
    def body(x_ref, y_ref, o_ref):
        o_ref[...] = lax.dot_general(
            x_ref[0], y_ref[0], (((1,), (1,)), ((), ())),
            preferred_element_type=jnp.float32)

    return pl.pallas_call(
        body,
        grid=(GB, GB),
        in_specs=[pl.BlockSpec((1, BM, 64), lambda i, j: (0, i, 0)),
                  pl.BlockSpec((1, BM, 64), lambda i, j: (1, j, 0))],
        out_specs=pl.BlockSpec((BM, BM), lambda i, j: (i, j)),
        out_shape=jax.ShapeDtypeStruct((NPAD, NPAD), jnp.float32),
    )(feats, feats)


def kernel(edge_index_m, edge_index_d, data_m, data_d, x_m, x_d, beta1, beta2,
           gcn_x1_W, gcn_x1_b, gcn_x2_W, gcn_x2_b, gcn_x3_W, gcn_x3_b,
           gcn_y1_W, gcn_y1_b, gcn_y2_W, gcn_y2_b, gcn_y3_W, gcn_y3_b,
           gat_x_W, gat_x_as, gat_x_ad, gat_x_b,
           gat_y_W, gat_y_as, gat_y_ad, gat_y_b,
           lin_x1_W, lin_x1_b, lin_x2_W, lin_x2_b, lin_x3_W, lin_x3_b,
           lin_y1_W, lin_y1_b, lin_y2_W, lin_y2_b, lin_y3_W, lin_y3_b):
    ei_m = edge_index_m.astype(jnp.int32)
    ei_d = edge_index_d.astype(jnp.int32)

    selfn = jnp.arange(NPAD, dtype=jnp.int32)
    src = jnp.stack([jnp.concatenate([ei_m[0], selfn]),
                     jnp.concatenate([ei_d[0], selfn])])
    dst = jnp.stack([jnp.concatenate([ei_m[1], selfn]),
                     jnp.concatenate([ei_d[1], selfn])])
    flat = jnp.stack([ei_m[0] * NN + ei_m[1], ei_d[0] * NN + ei_d[1]])

    flat_k1 = flat.reshape(2, NT, K1C, CH)
    src_k1 = src[:, :NE].reshape(2, NT, K1C, CH)
    dst_k1 = dst[:, :NE].reshape(2, NT, K1C, CH)

    buck = dst // RPT
    perm = jnp.argsort(buck, axis=1).astype(jnp.int32)
    bsort = jnp.take_along_axis(buck, perm, axis=1)
    tt = jnp.arange(NT, dtype=jnp.int32)
    starts = jax.vmap(lambda bs: jnp.searchsorted(bs, tt, side="left")
                      )(bsort).astype(jnp.int32)
    ends = jax.vmap(lambda bs: jnp.searchsorted(bs, tt, side="right")
                    )(bsort).astype(jnp.int32)
    seglen = ends - starts
    pos = starts[:, :, None] + jnp.arange(CAP, dtype=jnp.int32)[None, None, :]
    pos = jnp.minimum(pos, EP - 1)
    permp = jnp.take_along_axis(perm, pos.reshape(2, -1), axis=1
                                ).reshape(2, NT, CAP)
    permp = permp + (jnp.arange(2, dtype=jnp.int32) * EP)[:, None, None]
    permp = permp[:, :, None, :]
    nchunks = ((seglen + (CH - 1)) // CH)[:, None, :]
    nchunks64 = ((seglen + 63) // 64)[:, None, :]
    seglen = seglen[:, None, :]

    coeff = _sc_prep(data_m.reshape(-1), data_d.reshape(-1),
                     flat_k1, src_k1, dst_k1)
    psrc, pdst, pcoef, pval = _sc_permute(
        src.reshape(-1), dst.reshape(-1), coeff.reshape(-1), permp, seglen)

    pad = ((0, NPAD - NN), (0, 0))
    xpad = jnp.stack([jnp.pad(x_m, pad), jnp.pad(x_d, pad)])

    w1 = jnp.stack([gcn_x1_W, gcn_y1_W])
    b1 = jnp.stack([gcn_x1_b, gcn_y1_b])[:, None, :]
    w2 = jnp.stack([gcn_x2_W, gcn_y2_W])
    b2 = jnp.stack([gcn_x2_b, gcn_y2_b])[:, None, :]
    w3 = jnp.stack([gcn_x3_W, gcn_y3_W])
    b3 = jnp.stack([gcn_x3_b, gcn_y3_b])[:, None, :]
    wg = jnp.stack([gat_x_W, gat_y_W])
    bg = jnp.stack([gat_x_b, gat_y_b])[:, None, :]

    rep = jnp.repeat(jnp.eye(NHEAD, dtype=jnp.float32), F // NHEAD, axis=0)
    amat_s = jnp.stack([gat_x_as.reshape(-1)[:, None] * rep,
                        gat_y_as.reshape(-1)[:, None] * rep])
    amat_d = jnp.stack([gat_x_ad.reshape(-1)[:, None] * rep,
                        gat_y_ad.reshape(-1)[:, None] * rep])
    rmat = jnp.concatenate(
        [jnp.repeat(jnp.eye(NHEAD, dtype=jnp.float32), F // NHEAD, axis=1),
         jnp.zeros((16 - NHEAD, F), jnp.float32)], axis=0)

    xw1 = _tc_mm(xpad, w1)
    acc1 = _sc_prop(xw1.reshape(2 * NPAD, F), psrc, pdst, pcoef, nchunks)
    _, xwg = _tc_lrelu_mm(acc1, b1, wg)
    as_t = _tc_mm(xwg, amat_s)
    ad_t = _tc_mm(xwg, amat_d)
    featx = jnp.concatenate(
        [xwg, as_t, jnp.zeros((2, NPAD, 124), jnp.float32)],
        axis=2).reshape(2 * NPAD, F + 128)
    adfull = jnp.concatenate(
        [ad_t, jnp.zeros((2, NPAD, 124), jnp.float32)],
        axis=2).reshape(2 * NPAD, 128)
    accg, zg = _sc_gat(featx, adfull, psrc, pdst, pval, nchunks64)
    zg = zg.reshape(2, NPAD, 16)
    x1g, xw2 = _tc_gat_post(accg, zg, bg, w2, rmat)
    acc2 = _sc_prop(xw2.reshape(2 * NPAD, F), psrc, pdst, pcoef, nchunks)
    x2, xw3 = _tc_lrelu_mm(acc2, b2, w3)
    acc3 = _sc_prop(xw3.reshape(2 * NPAD, F), psrc, pdst, pcoef, nchunks)

    betas = jnp.stack([beta1, beta2])
    l1 = jnp.stack([lin_x1_W, lin_y1_W])
    c1 = jnp.stack([lin_x1_b, lin_y1_b])[:, None, :]
    l2 = jnp.stack([lin_x2_W, lin_y2_W])
    c2 = jnp.stack([lin_x2_b, lin_y2_b])[:, None, :]
    l3 = jnp.stack([lin_x3_W, lin_y3_W])
    c3 = jnp.stack([lin_x3_b, lin_y3_b])[:, None, :]
    feats = _tc_mlp(acc3, b3, x1g, x2, betas, l1, c1, l2, c2, l3, c3)
    return _tc_final(feats)[:NN, :NN]


# speedup vs baseline: 6.1107x; 1.0162x over previous
import functools

import jax
import jax.numpy as jnp
from jax import lax
from jax.experimental import pallas as pl
from jax.experimental.pallas import tpu as pltpu
from jax.experimental.pallas import tpu_sc as plsc

NN = 5000
NPAD = 5120
NE = 160000
EP = NE + NPAD
NT = 16
CH = 80
K1C = (NE // NT) // CH
RPT = NPAD // NT
F = 256
NHEAD = 4
FZ = F + 16
CAP = 16000
CAPC = CAP // CH

_MESH = dict(core_axis_name="c", subcore_axis_name="s")


def _lr(v):
    return jnp.where(v > 0, v, v * 0.01)


def _qrsqrt(x):
    i = lax.bitcast_convert_type(x, jnp.int32)
    i = jnp.int32(0x5F3759DF) - lax.shift_right_logical(i, 1)
    y = lax.bitcast_convert_type(i, jnp.float32)
    for _ in range(3):
        y = y * (1.5 - 0.5 * x * y * y)
    return y


def _sc_prep(data_m_flat, data_d_flat, flat_k1, src_k1, dst_k1):
    mesh = plsc.VectorSubcoreMesh(**_MESH)

    @functools.partial(
        pl.kernel,
        out_type=jax.ShapeDtypeStruct((2, 1, EP), jnp.float32),
        mesh=mesh,
        compiler_params=pltpu.CompilerParams(needs_layout_passes=False),
        scratch_types=[
            pltpu.VMEM((K1C, CH), jnp.int32),
            pltpu.VMEM((K1C, CH), jnp.int32),
            pltpu.VMEM((K1C, CH), jnp.float32),
            pltpu.VMEM((RPT,), jnp.float32),
            pltpu.VMEM((NPAD,), jnp.float32),
            pltpu.VMEM_SHARED((NPAD,), jnp.float32),
            pltpu.SemaphoreType.DMA,
        ],
    )
    def k(dm_ref, dd_ref, flat_ref, src_ref, dst_ref, coef_ref,
          fbuf, dbuf, wbuf, dv, dinvtab, deg_sh, sem):
        c = lax.axis_index("c")
        s = lax.axis_index("s")
        pltpu.sync_copy(flat_ref.at[c, s], fbuf)
        pltpu.sync_copy(dst_ref.at[c, s], dbuf)
        for j in range(RPT // 16):
            dv[pl.ds(j * 16, 16)] = jnp.full((16,), 1.0, jnp.float32)
        pltpu.sync_copy(dv, deg_sh.at[pl.ds(s * RPT, RPT)])
        plsc.subcore_barrier()

        def gather_all(dref):
            def chunk(kk, _):
                pltpu.async_copy(dref.at[fbuf.at[kk]], wbuf.at[kk], sem).wait()
                return 0
            lax.fori_loop(0, K1C, chunk, 0)

        @pl.when(c == 0)
        def _():
            gather_all(dm_ref)

        @pl.when(c == 1)
        def _():
            gather_all(dd_ref)

        def degadd(kk, _):
            pltpu.sync_copy(wbuf.at[kk], deg_sh.at[dbuf.at[kk]], add=True)
            return 0
        lax.fori_loop(0, K1C, degadd, 0)
        plsc.subcore_barrier()

        pltpu.sync_copy(deg_sh.at[pl.ds(s * RPT, RPT)], dv)
        for j in range(RPT // 16):
            dv[pl.ds(j * 16, 16)] = _qrsqrt(dv[pl.ds(j * 16, 16)])
        pltpu.sync_copy(dv, deg_sh.at[pl.ds(s * RPT, RPT)])
        for j in range(RPT // 16):
            y = dv[pl.ds(j * 16, 16)]
            dv[pl.ds(j * 16, 16)] = y * y
        pltpu.sync_copy(dv, coef_ref.at[c, 0].at[pl.ds(NE + s * RPT, RPT)])
        plsc.subcore_barrier()

        pltpu.sync_copy(deg_sh, dinvtab)
        pltpu.sync_copy(src_ref.at[c, s], fbuf)

        def normchunk(kk, _):
            def sub(i, _):
                sl = pl.ds(i * 16, 16)
                sv = fbuf[kk, sl]
                tv = dbuf[kk, sl]
                dsv = plsc.load_gather(dinvtab, [sv])
                dtv = plsc.load_gather(dinvtab, [tv])
                wbuf[kk, sl] = dsv * wbuf[kk, sl] * dtv
                return 0
            lax.fori_loop(0, CH // 16, sub, 0)
            pltpu.sync_copy(
                wbuf.at[kk],
                coef_ref.at[c, 0].at[pl.ds(s * (K1C * CH) + kk * CH, CH)])
            return 0
        lax.fori_loop(0, K1C, normchunk, 0)

    return k(data_m_flat, data_d_flat, flat_k1, src_k1, dst_k1)


def _sc_permute(src_all, dst_all, coeff, permp, seglen):
    mesh = plsc.VectorSubcoreMesh(**_MESH)

    @functools.partial(
        pl.kernel,
        out_type=(
            jax.ShapeDtypeStruct((2, NT, 1, CAP), jnp.int32),
            jax.ShapeDtypeStruct((2, NT, 1, CAP), jnp.int32),
            jax.ShapeDtypeStruct((2, NT, 1, CAP), jnp.float32),
            jax.ShapeDtypeStruct((2, NT, 1, CAP), jnp.float32),
        ),
        mesh=mesh,
        compiler_params=pltpu.CompilerParams(needs_layout_passes=False),
        scratch_types=[
            pltpu.VMEM((CH,), jnp.int32),
            pltpu.VMEM((CH,), jnp.int32),
            pltpu.VMEM((CH,), jnp.float32),
            pltpu.VMEM((1, 16), jnp.int32),
            pltpu.SemaphoreType.DMA,
        ],
    )
    def k(src_ref, dst_ref, coef_ref, perm_ref, len_ref,
          psrc_ref, pdst_ref, pcoef_ref, pval_ref, pbuf, ibuf, cbuf, lbuf,
          sem):
        c = lax.axis_index("c")
        s = lax.axis_index("s")
        pltpu.sync_copy(len_ref.at[c], lbuf)
        seg = plsc.load_gather(
            lbuf, [jnp.zeros((16,), jnp.int32),
                   jnp.full((16,), s, jnp.int32)])[0]
        iota = lax.iota(jnp.int32, 16)

        def chunk(kk, _):
            pltpu.sync_copy(perm_ref.at[c, s, 0].at[pl.ds(kk * CH, CH)], pbuf)
            base = kk * CH
            pltpu.async_copy(src_ref.at[pbuf], ibuf, sem).wait()
            for i in range(CH // 16):
                sl = pl.ds(i * 16, 16)
                ibuf[sl] = ibuf[sl] + c * NPAD
            pltpu.sync_copy(ibuf, psrc_ref.at[c, s, 0].at[pl.ds(base, CH)])
            pltpu.async_copy(dst_ref.at[pbuf], ibuf, sem).wait()
            for i in range(CH // 16):
                sl = pl.ds(i * 16, 16)
                valid = (base + i * 16 + iota) < seg
                ibuf[sl] = jnp.where(valid, ibuf[sl] - s * RPT, 0)
            pltpu.sync_copy(ibuf, pdst_ref.at[c, s, 0].at[pl.ds(base, CH)])
            pltpu.async_copy(coef_ref.at[pbuf], cbuf, sem).wait()
            for i in range(CH // 16):
                sl = pl.ds(i * 16, 16)
                valid = (base + i * 16 + iota) < seg
                cbuf[sl] = jnp.where(valid, cbuf[sl], 0.0)
            pltpu.sync_copy(cbuf, pcoef_ref.at[c, s, 0].at[pl.ds(base, CH)])
            for i in range(CH // 16):
                sl = pl.ds(i * 16, 16)
                valid = (base + i * 16 + iota) < seg
                cbuf[sl] = jnp.where(valid, 1.0, 0.0)
            pltpu.sync_copy(cbuf, pval_ref.at[c, s, 0].at[pl.ds(base, CH)])
            return 0
        nch = lax.div(seg + (CH - 1), CH)
        lax.fori_loop(0, nch, chunk, 0)
        for i in range(CH // 16):
            ibuf[pl.ds(i * 16, 16)] = jnp.zeros((16,), jnp.int32)
            cbuf[pl.ds(i * 16, 16)] = jnp.zeros((16,), jnp.float32)

        def padchunk(kk, _):
            base = kk * CH
            pltpu.sync_copy(ibuf, psrc_ref.at[c, s, 0].at[pl.ds(base, CH)])
            pltpu.sync_copy(ibuf, pdst_ref.at[c, s, 0].at[pl.ds(base, CH)])
            pltpu.sync_copy(cbuf, pcoef_ref.at[c, s, 0].at[pl.ds(base, CH)])
            pltpu.sync_copy(cbuf, pval_ref.at[c, s, 0].at[pl.ds(base, CH)])
            return 0
        lax.fori_loop(nch, CAPC, padchunk, 0)

    return k(src_all, dst_all, coeff, permp, seglen)


def _sc_prop(feat2d, pk, nchunks):
    mesh = plsc.VectorSubcoreMesh(**_MESH)

    @functools.partial(
        pl.kernel,
        out_type=jax.ShapeDtypeStruct((2, NPAD, F), jnp.float32),
        mesh=mesh,
        compiler_params=pltpu.CompilerParams(needs_layout_passes=False),
        scratch_types=[
            pltpu.VMEM((3 * CH,), jnp.int32),
            pltpu.VMEM((3 * CH,), jnp.int32),
            pltpu.VMEM((CH, F), jnp.float32),
            pltpu.VMEM((CH, F), jnp.float32),
            pltpu.VMEM((RPT, F), jnp.float32),
            pltpu.VMEM((1, 16), jnp.int32),
            pltpu.SemaphoreType.DMA,
            pltpu.SemaphoreType.DMA,
        ],
    )
    def k(feat_ref, pk_ref, nch_ref, out_ref,
          pka, pkb, rowsa, rowsb, acc, lbuf, sema, semb):
        c = lax.axis_index("c")
        s = lax.axis_index("s")
        pltpu.sync_copy(nch_ref.at[c], lbuf)
        nch = plsc.load_gather(
            lbuf, [jnp.zeros((16,), jnp.int32),
                   jnp.full((16,), s, jnp.int32)])[0]

        def zr(r, _):
            for j in range(F // 16):
                acc[r, pl.ds(j * 16, 16)] = jnp.zeros((16,), jnp.float32)
            return 0
        lax.fori_loop(0, RPT, zr, 0)

        def load_fire(kk, pkx, rowsx, semx):
            pltpu.sync_copy(pk_ref.at[c, s, 0].at[pl.ds(kk * (3 * CH),
                                                        3 * CH)], pkx)
            pltpu.async_copy(feat_ref.at[pkx.at[pl.ds(0, CH)]], rowsx, semx)

        def waitx(rowsx, semx):
            pltpu.make_async_copy(feat_ref.at[pl.ds(0, CH), :], rowsx,
                                  semx).wait()

        def process(pkx, rowsx):
            def grp(i, _):
                dvec = pkx[pl.ds(CH + i * 16, 16)]
                cvec = lax.bitcast_convert_type(
                    pkx[pl.ds(2 * CH + i * 16, 16)], jnp.float32)
                for e in range(16):
                    cc = cvec[e]
                    dl = dvec[e]
                    r = i * 16 + e
                    for j in range(F // 16):
                        fs = pl.ds(j * 16, 16)
                        acc[dl, fs] = acc[dl, fs] + rowsx[r, fs] * cc
                return 0
            lax.fori_loop(0, CH // 16, grp, 0)

        load_fire(0, pka, rowsa, sema)

        def pair(kk2, _):
            k0 = 2 * kk2
            k1 = k0 + 1

            @pl.when(k1 < nch)
            def _():
                load_fire(k1, pkb, rowsb, semb)
            waitx(rowsa, sema)
            process(pka, rowsa)

            @pl.when(k1 + 1 < nch)
            def _():
                load_fire(k1 + 1, pka, rowsa, sema)

            @pl.when(k1 < nch)
            def _():
                waitx(rowsb, semb)
                process(pkb, rowsb)
            return 0
        lax.fori_loop(0, (nch + 1) // 2, pair, 0)
        pltpu.sync_copy(acc, out_ref.at[c, pl.ds(s * RPT, RPT), :])

    return k(feat2d, pk, nchunks)


def _sc_gat(featx, adfull, psrc, pdst, pval, nchunks):
    mesh = plsc.VectorSubcoreMesh(**_MESH)
    CHG = 64

    @functools.partial(
        pl.kernel,
        out_type=(jax.ShapeDtypeStruct((2, NPAD, F), jnp.float32),
                  jax.ShapeDtypeStruct((2, NT, 1, RPT * 16), jnp.float32)),
        mesh=mesh,
        compiler_params=pltpu.CompilerParams(needs_layout_passes=False),
        scratch_types=[
            pltpu.VMEM((CHG,), jnp.int32),
            pltpu.VMEM((CHG,), jnp.int32),
            pltpu.VMEM((CHG,), jnp.float32),
            pltpu.VMEM((CHG,), jnp.int32),
            pltpu.VMEM((CHG, F + 128), jnp.float32),
            pltpu.VMEM((CHG, 128), jnp.float32),
            pltpu.VMEM((RPT, F), jnp.float32),
            pltpu.VMEM((RPT * 16,), jnp.float32),
            pltpu.VMEM((1, 16), jnp.int32),
            pltpu.SemaphoreType.DMA,
            pltpu.SemaphoreType.DMA,
        ],
    )
    def k(feat_ref, ad_ref, src_ref, dst_ref, val_ref, nch_ref,
          out_ref, z_ref,
          sbuf, dbuf, vbuf, gbuf, rows, adrows, acc, accz, lbuf, sem, sem2):
        c = lax.axis_index("c")
        s = lax.axis_index("s")
        pltpu.sync_copy(nch_ref.at[c], lbuf)
        nch = plsc.load_gather(
            lbuf, [jnp.zeros((16,), jnp.int32),
                   jnp.full((16,), s, jnp.int32)])[0]

        def zr(r, _):
            for j in range(F // 16):
                acc[r, pl.ds(j * 16, 16)] = jnp.zeros((16,), jnp.float32)
            accz[pl.ds(r * 16, 16)] = jnp.zeros((16,), jnp.float32)
            return 0
        lax.fori_loop(0, RPT, zr, 0)

        goff = c * NPAD + s * RPT

        def chunk(kk, _):
            base = kk * CHG
            pltpu.sync_copy(src_ref.at[c, s, 0].at[pl.ds(base, CHG)], sbuf)
            pltpu.sync_copy(dst_ref.at[c, s, 0].at[pl.ds(base, CHG)], dbuf)
            pltpu.sync_copy(val_ref.at[c, s, 0].at[pl.ds(base, CHG)], vbuf)
            for i in range(CHG // 16):
                sl = pl.ds(i * 16, 16)
                gbuf[sl] = dbuf[sl] + goff
            cp1 = pltpu.async_copy(feat_ref.at[sbuf], rows, sem)
            cp2 = pltpu.async_copy(ad_ref.at[gbuf], adrows, sem2)
            cp1.wait()
            cp2.wait()

            def grp(i, _):
                sl = pl.ds(i * 16, 16)
                dvec = dbuf[sl]
                vvec = vbuf[sl]
                for e in range(16):
                    dl = dvec[e]
                    r = i * 16 + e
                    x = rows[r, pl.ds(F, 16)] + adrows[r, pl.ds(0, 16)]
                    x = jnp.maximum(x, x * 0.2)
                    pv = jnp.exp(x) * vvec[e]
                    for h in range(NHEAD):
                        ph = pv[h]
                        for j in range(F // (16 * NHEAD)):
                            fs = pl.ds(h * (F // NHEAD) + j * 16, 16)
                            acc[dl, fs] = acc[dl, fs] + rows[r, fs] * ph
                    accz[pl.ds(dl * 16, 16)] = accz[pl.ds(dl * 16, 16)] + pv
                return 0
            lax.fori_loop(0, CHG // 16, grp, 0)
            return 0
        lax.fori_loop(0, nch, chunk, 0)
        pltpu.sync_copy(acc, out_ref.at[c, pl.ds(s * RPT, RPT), :])
        pltpu.sync_copy(accz, z_ref.at[c, s, 0])

    return k(featx, adfull, psrc, pdst, pval, nchunks)


BM = 512
GB = NPAD // BM


def _tc_mm(x, w):
    ki, ko = w.shape[1], w.shape[2]

    def body(x_ref, w_ref, o_ref):
        o_ref[...] = jnp.dot(x_ref[0], w_ref[0],
                             preferred_element_type=jnp.float32)[None]

    return pl.pallas_call(
        body,
        grid=(2, GB),
        in_specs=[pl.BlockSpec((1, BM, ki), lambda g, i: (g, i, 0)),
                  pl.BlockSpec((1, ki, ko), lambda g, i: (g, 0, 0))],
        out_specs=pl.BlockSpec((1, BM, ko), lambda g, i: (g, i, 0)),
        out_shape=jax.ShapeDtypeStruct((2, NPAD, ko), jnp.float32),
    )(x, w)


def _tc_lrelu_mm(acc, b, w):
    ko = w.shape[2]

    def body(a_ref, b_ref, w_ref, x_ref, xw_ref):
        X = _lr(a_ref[0] + b_ref[0])
        x_ref[...] = X[None]
        xw_ref[...] = jnp.dot(X, w_ref[0],
                              preferred_element_type=jnp.float32)[None]

    return pl.pallas_call(
        body,
        grid=(2, GB),
        in_specs=[pl.BlockSpec((1, BM, F), lambda g, i: (g, i, 0)),
                  pl.BlockSpec((1, 1, F), lambda g, i: (g, 0, 0)),
                  pl.BlockSpec((1, F, ko), lambda g, i: (g, 0, 0))],
        out_specs=[pl.BlockSpec((1, BM, F), lambda g, i: (g, i, 0)),
                   pl.BlockSpec((1, BM, ko), lambda g, i: (g, i, 0))],
        out_shape=[jax.ShapeDtypeStruct((2, NPAD, F), jnp.float32),
                   jax.ShapeDtypeStruct((2, NPAD, ko), jnp.float32)],
    )(acc, b, w)


def _tc_gat_post(acc, zz, bg, w2, rmat):
    def body(a_ref, z_ref, bg_ref, w_ref, r_ref, x_ref, xw_ref):
        zr = jnp.dot(z_ref[0], r_ref[...], preferred_element_type=jnp.float32)
        X = _lr(a_ref[0] / jnp.maximum(zr, 1e-16) + bg_ref[0])
        x_ref[...] = X[None]
        xw_ref[...] = jnp.dot(X, w_ref[0],
                              preferred_element_type=jnp.float32)[None]

    return pl.pallas_call(
        body,
        grid=(2, GB),
        in_specs=[pl.BlockSpec((1, BM, F), lambda g, i: (g, i, 0)),
                  pl.BlockSpec((1, BM, 16), lambda g, i: (g, i, 0)),
                  pl.BlockSpec((1, 1, F), lambda g, i: (g, 0, 0)),
                  pl.BlockSpec((1, F, F), lambda g, i: (g, 0, 0)),
                  pl.BlockSpec((16, F), lambda g, i: (0, 0))],
        out_specs=[pl.BlockSpec((1, BM, F), lambda g, i: (g, i, 0)),
                   pl.BlockSpec((1, BM, F), lambda g, i: (g, i, 0))],
        out_shape=[jax.ShapeDtypeStruct((2, NPAD, F), jnp.float32),
                   jax.ShapeDtypeStruct((2, NPAD, F), jnp.float32)],
    )(acc, zz, bg, w2, rmat)


def _tc_mlp(acc3, b3, x1g, x2, betas, l1, c1, l2, c2, l3, c3):
    def body(a_ref, b_ref, x1_ref, x2_ref, bt_ref,
             l1_ref, c1_ref, l2_ref, c2_ref, l3_ref, c3_ref, o_ref):
        g = pl.program_id(0)
        b0 = bt_ref[g, 0]
        b1 = bt_ref[g, 1]
        X3 = _lr(a_ref[0] + b_ref[0])
        X = b0 * x1_ref[0] + b1 * x2_ref[0] + (1.0 - b0 - b1) * X3
        h = _lr(jnp.dot(X, l1_ref[0], preferred_element_type=jnp.float32)
                + c1_ref[0])
        h = _lr(jnp.dot(h, l2_ref[0], preferred_element_type=jnp.float32)
                + c2_ref[0])
        h = _lr(jnp.dot(h, l3_ref[0], preferred_element_type=jnp.float32)
                + c3_ref[0])
        o_ref[...] = h[None]

    return pl.pallas_call(
        body,
        grid=(2, GB),
        in_specs=[pl.BlockSpec((1, BM, F), lambda g, i: (g, i, 0)),
                  pl.BlockSpec((1, 1, F), lambda g, i: (g, 0, 0)),
                  pl.BlockSpec((1, BM, F), lambda g, i: (g, i, 0)),
                  pl.BlockSpec((1, BM, F), lambda g, i: (g, i, 0)),
                  pl.BlockSpec(memory_space=pltpu.SMEM),
                  pl.BlockSpec((1, F, F), lambda g, i: (g, 0, 0)),
                  pl.BlockSpec((1, 1, F), lambda g, i: (g, 0, 0)),
                  pl.BlockSpec((1, F, 128), lambda g, i: (g, 0, 0)),
                  pl.BlockSpec((1, 1, 128), lambda g, i: (g, 0, 0)),
                  pl.BlockSpec((1, 128, 64), lambda g, i: (g, 0, 0)),
                  pl.BlockSpec((1, 1, 64), lambda g, i: (g, 0, 0))],
        out_specs=pl.BlockSpec((1, BM, 64), lambda g, i: (g, i, 0)),
        out_shape=jax.ShapeDtypeStruct((2, NPAD, 64), jnp.float32),
    )(acc3, b3, x1g, x2, betas, l1, c1, l2, c2, l3, c3)


def _tc_final(feats):
    def body(x_ref, y_ref, o_ref):
        o_ref[...] = lax.dot_general(
            x_ref[0], y_ref[0], (((1,), (1,)), ((), ())),
            preferred_element_type=jnp.float32)

    return pl.pallas_call(
        body,
        grid=(GB, GB),
        in_specs=[pl.BlockSpec((1, BM, 64), lambda i, j: (0, i, 0)),
                  pl.BlockSpec((1, BM, 64), lambda i, j: (1, j, 0))],
        out_specs=pl.BlockSpec((BM, BM), lambda i, j: (i, j)),
        out_shape=jax.ShapeDtypeStruct((NPAD, NPAD), jnp.float32),
    )(feats, feats)


def kernel(edge_index_m, edge_index_d, data_m, data_d, x_m, x_d, beta1, beta2,
           gcn_x1_W, gcn_x1_b, gcn_x2_W, gcn_x2_b, gcn_x3_W, gcn_x3_b,
           gcn_y1_W, gcn_y1_b, gcn_y2_W, gcn_y2_b, gcn_y3_W, gcn_y3_b,
           gat_x_W, gat_x_as, gat_x_ad, gat_x_b,
           gat_y_W, gat_y_as, gat_y_ad, gat_y_b,
           lin_x1_W, lin_x1_b, lin_x2_W, lin_x2_b, lin_x3_W, lin_x3_b,
           lin_y1_W, lin_y1_b, lin_y2_W, lin_y2_b, lin_y3_W, lin_y3_b):
    ei_m = edge_index_m.astype(jnp.int32)
    ei_d = edge_index_d.astype(jnp.int32)

    selfn = jnp.arange(NPAD, dtype=jnp.int32)
    src = jnp.stack([jnp.concatenate([ei_m[0], selfn]),
                     jnp.concatenate([ei_d[0], selfn])])
    dst = jnp.stack([jnp.concatenate([ei_m[1], selfn]),
                     jnp.concatenate([ei_d[1], selfn])])
    flat = jnp.stack([ei_m[0] * NN + ei_m[1], ei_d[0] * NN + ei_d[1]])

    flat_k1 = flat.reshape(2, NT, K1C, CH)
    src_k1 = src[:, :NE].reshape(2, NT, K1C, CH)
    dst_k1 = dst[:, :NE].reshape(2, NT, K1C, CH)

    buck = dst // RPT
    perm = jnp.argsort(buck, axis=1).astype(jnp.int32)
    bsort = jnp.take_along_axis(buck, perm, axis=1)
    tt = jnp.arange(NT, dtype=jnp.int32)
    starts = jax.vmap(lambda bs: jnp.searchsorted(bs, tt, side="left")
                      )(bsort).astype(jnp.int32)
    ends = jax.vmap(lambda bs: jnp.searchsorted(bs, tt, side="right")
                    )(bsort).astype(jnp.int32)
    seglen = ends - starts
    pos = starts[:, :, None] + jnp.arange(CAP, dtype=jnp.int32)[None, None, :]
    pos = jnp.minimum(pos, EP - 1)
    permp = jnp.take_along_axis(perm, pos.reshape(2, -1), axis=1
                                ).reshape(2, NT, CAP)
    permp = permp + (jnp.arange(2, dtype=jnp.int32) * EP)[:, None, None]
    permp = permp[:, :, None, :]
    nchunks = ((seglen + (CH - 1)) // CH)[:, None, :]
    nchunks64 = ((seglen + 63) // 64)[:, None, :]
    seglen = seglen[:, None, :]

    coeff = _sc_prep(data_m.reshape(-1), data_d.reshape(-1),
                     flat_k1, src_k1, dst_k1)
    psrc, pdst, pcoef, pval = _sc_permute(
        src.reshape(-1), dst.reshape(-1), coeff.reshape(-1), permp, seglen)

    pad = ((0, NPAD - NN), (0, 0))
    xpad = jnp.stack([jnp.pad(x_m, pad), jnp.pad(x_d, pad)])

    w1 = jnp.stack([gcn_x1_W, gcn_y1_W])
    b1 = jnp.stack([gcn_x1_b, gcn_y1_b])[:, None, :]
    w2 = jnp.stack([gcn_x2_W, gcn_y2_W])
    b2 = jnp.stack([gcn_x2_b, gcn_y2_b])[:, None, :]
    w3 = jnp.stack([gcn_x3_W, gcn_y3_W])
    b3 = jnp.stack([gcn_x3_b, gcn_y3_b])[:, None, :]
    wg = jnp.stack([gat_x_W, gat_y_W])
    bg = jnp.stack([gat_x_b, gat_y_b])[:, None, :]

    rep = jnp.repeat(jnp.eye(NHEAD, dtype=jnp.float32), F // NHEAD, axis=0)
    amat_s = jnp.stack([gat_x_as.reshape(-1)[:, None] * rep,
                        gat_y_as.reshape(-1)[:, None] * rep])
    amat_d = jnp.stack([gat_x_ad.reshape(-1)[:, None] * rep,
                        gat_y_ad.reshape(-1)[:, None] * rep])
    rmat = jnp.concatenate(
        [jnp.repeat(jnp.eye(NHEAD, dtype=jnp.float32), F // NHEAD, axis=1),
         jnp.zeros((16 - NHEAD, F), jnp.float32)], axis=0)

    pk = jnp.stack(
        [psrc.reshape(2, NT, 1, CAP // CH, CH),
         pdst.reshape(2, NT, 1, CAP // CH, CH),
         lax.bitcast_convert_type(pcoef, jnp.int32
                                  ).reshape(2, NT, 1, CAP // CH, CH)],
        axis=4).reshape(2, NT, 1, 3 * CAP)

    xw1 = _tc_mm(xpad, w1)
    acc1 = _sc_prop(xw1.reshape(2 * NPAD, F), pk, nchunks)
    _, xwg = _tc_lrelu_mm(acc1, b1, wg)
    as_t = _tc_mm(xwg, amat_s)
    ad_t = _tc_mm(xwg, amat_d)
    featx = jnp.concatenate(
        [xwg, as_t, jnp.zeros((2, NPAD, 124), jnp.float32)],
        axis=2).reshape(2 * NPAD, F + 128)
    adfull = jnp.concatenate(
        [ad_t, jnp.zeros((2, NPAD, 124), jnp.float32)],
        axis=2).reshape(2 * NPAD, 128)
    accg, zg = _sc_gat(featx, adfull, psrc, pdst, pval, nchunks64)
    zg = zg.reshape(2, NPAD, 16)
    x1g, xw2 = _tc_gat_post(accg, zg, bg, w2, rmat)
    acc2 = _sc_prop(xw2.reshape(2 * NPAD, F), pk, nchunks)
    x2, xw3 = _tc_lrelu_mm(acc2, b2, w3)
    acc3 = _sc_prop(xw3.reshape(2 * NPAD, F), pk, nchunks)

    betas = jnp.stack([beta1, beta2])
    l1 = jnp.stack([lin_x1_W, lin_y1_W])
    c1 = jnp.stack([lin_x1_b, lin_y1_b])[:, None, :]
    l2 = jnp.stack([lin_x2_W, lin_y2_W])
    c2 = jnp.stack([lin_x2_b, lin_y2_b])[:, None, :]
    l3 = jnp.stack([lin_x3_W, lin_y3_W])
    c3 = jnp.stack([lin_x3_b, lin_y3_b])[:, None, :]
    feats = _tc_mlp(acc3, b3, x1g, x2, betas, l1, c1, l2, c2, l3, c3)
    return _tc_final(feats)[:NN, :NN]


# speedup vs baseline: 6.1233x; 1.0021x over previous
import functools

import jax
import jax.numpy as jnp
from jax import lax
from jax.experimental import pallas as pl
from jax.experimental.pallas import tpu as pltpu
from jax.experimental.pallas import tpu_sc as plsc

NN = 5000
NPAD = 5120
NE = 160000
EP = NE + NPAD
NT = 16
CH = 80
K1C = (NE // NT) // CH
RPT = NPAD // NT
F = 256
NHEAD = 4
FZ = F + 16
CAP = 16000
CAPC = CAP // CH

_MESH = dict(core_axis_name="c", subcore_axis_name="s")


def _lr(v):
    return jnp.where(v > 0, v, v * 0.01)


def _qrsqrt(x):
    i = lax.bitcast_convert_type(x, jnp.int32)
    i = jnp.int32(0x5F3759DF) - lax.shift_right_logical(i, 1)
    y = lax.bitcast_convert_type(i, jnp.float32)
    for _ in range(3):
        y = y * (1.5 - 0.5 * x * y * y)
    return y


def _sc_prep(data_m_flat, data_d_flat, flat_k1, src_k1, dst_k1):
    mesh = plsc.VectorSubcoreMesh(**_MESH)

    @functools.partial(
        pl.kernel,
        out_type=jax.ShapeDtypeStruct((2, 1, EP), jnp.float32),
        mesh=mesh,
        compiler_params=pltpu.CompilerParams(needs_layout_passes=False),
        scratch_types=[
            pltpu.VMEM((K1C, CH), jnp.int32),
            pltpu.VMEM((K1C, CH), jnp.int32),
            pltpu.VMEM((K1C, CH), jnp.float32),
            pltpu.VMEM((RPT,), jnp.float32),
            pltpu.VMEM((NPAD,), jnp.float32),
            pltpu.VMEM_SHARED((NPAD,), jnp.float32),
            pltpu.SemaphoreType.DMA,
        ],
    )
    def k(dm_ref, dd_ref, flat_ref, src_ref, dst_ref, coef_ref,
          fbuf, dbuf, wbuf, dv, dinvtab, deg_sh, sem):
        c = lax.axis_index("c")
        s = lax.axis_index("s")
        pltpu.sync_copy(flat_ref.at[c, s], fbuf)
        pltpu.sync_copy(dst_ref.at[c, s], dbuf)
        for j in range(RPT // 16):
            dv[pl.ds(j * 16, 16)] = jnp.full((16,), 1.0, jnp.float32)
        pltpu.sync_copy(dv, deg_sh.at[pl.ds(s * RPT, RPT)])
        plsc.subcore_barrier()

        def gather_all(dref):
            def chunk(kk, _):
                pltpu.async_copy(dref.at[fbuf.at[kk]], wbuf.at[kk], sem).wait()
                return 0
            lax.fori_loop(0, K1C, chunk, 0)

        @pl.when(c == 0)
        def _():
            gather_all(dm_ref)

        @pl.when(c == 1)
        def _():
            gather_all(dd_ref)

        def degadd(kk, _):
            pltpu.sync_copy(wbuf.at[kk], deg_sh.at[dbuf.at[kk]], add=True)
            return 0
        lax.fori_loop(0, K1C, degadd, 0)
        plsc.subcore_barrier()

        pltpu.sync_copy(deg_sh.at[pl.ds(s * RPT, RPT)], dv)
        for j in range(RPT // 16):
            dv[pl.ds(j * 16, 16)] = _qrsqrt(dv[pl.ds(j * 16, 16)])
        pltpu.sync_copy(dv, deg_sh.at[pl.ds(s * RPT, RPT)])
        for j in range(RPT // 16):
            y = dv[pl.ds(j * 16, 16)]
            dv[pl.ds(j * 16, 16)] = y * y
        pltpu.sync_copy(dv, coef_ref.at[c, 0].at[pl.ds(NE + s * RPT, RPT)])
        plsc.subcore_barrier()

        pltpu.sync_copy(deg_sh, dinvtab)
        pltpu.sync_copy(src_ref.at[c, s], fbuf)

        def normchunk(kk, _):
            def sub(i, _):
                sl = pl.ds(i * 16, 16)
                sv = fbuf[kk, sl]
                tv = dbuf[kk, sl]
                dsv = plsc.load_gather(dinvtab, [sv])
                dtv = plsc.load_gather(dinvtab, [tv])
                wbuf[kk, sl] = dsv * wbuf[kk, sl] * dtv
                return 0
            lax.fori_loop(0, CH // 16, sub, 0)
            pltpu.sync_copy(
                wbuf.at[kk],
                coef_ref.at[c, 0].at[pl.ds(s * (K1C * CH) + kk * CH, CH)])
            return 0
        lax.fori_loop(0, K1C, normchunk, 0)

    return k(data_m_flat, data_d_flat, flat_k1, src_k1, dst_k1)


def _sc_permute(src_all, dst_all, coeff, permp, seglen):
    mesh = plsc.VectorSubcoreMesh(**_MESH)

    @functools.partial(
        pl.kernel,
        out_type=(
            jax.ShapeDtypeStruct((2, NT, 1, CAP), jnp.int32),
            jax.ShapeDtypeStruct((2, NT, 1, CAP), jnp.int32),
            jax.ShapeDtypeStruct((2, NT, 1, CAP), jnp.float32),
            jax.ShapeDtypeStruct((2, NT, 1, CAP), jnp.float32),
        ),
        mesh=mesh,
        compiler_params=pltpu.CompilerParams(needs_layout_passes=False),
        scratch_types=[
            pltpu.VMEM((CH,), jnp.int32),
            pltpu.VMEM((CH,), jnp.int32),
            pltpu.VMEM((CH,), jnp.float32),
            pltpu.VMEM((1, 16), jnp.int32),
            pltpu.SemaphoreType.DMA,
        ],
    )
    def k(src_ref, dst_ref, coef_ref, perm_ref, len_ref,
          psrc_ref, pdst_ref, pcoef_ref, pval_ref, pbuf, ibuf, cbuf, lbuf,
          sem):
        c = lax.axis_index("c")
        s = lax.axis_index("s")
        pltpu.sync_copy(len_ref.at[c], lbuf)
        seg = plsc.load_gather(
            lbuf, [jnp.zeros((16,), jnp.int32),
                   jnp.full((16,), s, jnp.int32)])[0]
        iota = lax.iota(jnp.int32, 16)

        def chunk(kk, _):
            pltpu.sync_copy(perm_ref.at[c, s, 0].at[pl.ds(kk * CH, CH)], pbuf)
            base = kk * CH
            pltpu.async_copy(src_ref.at[pbuf], ibuf, sem).wait()
            for i in range(CH // 16):
                sl = pl.ds(i * 16, 16)
                ibuf[sl] = ibuf[sl] + c * NPAD
            pltpu.sync_copy(ibuf, psrc_ref.at[c, s, 0].at[pl.ds(base, CH)])
            pltpu.async_copy(dst_ref.at[pbuf], ibuf, sem).wait()
            for i in range(CH // 16):
                sl = pl.ds(i * 16, 16)
                valid = (base + i * 16 + iota) < seg
                ibuf[sl] = jnp.where(valid, ibuf[sl] - s * RPT, 0)
            pltpu.sync_copy(ibuf, pdst_ref.at[c, s, 0].at[pl.ds(base, CH)])
            pltpu.async_copy(coef_ref.at[pbuf], cbuf, sem).wait()
            for i in range(CH // 16):
                sl = pl.ds(i * 16, 16)
                valid = (base + i * 16 + iota) < seg
                cbuf[sl] = jnp.where(valid, cbuf[sl], 0.0)
            pltpu.sync_copy(cbuf, pcoef_ref.at[c, s, 0].at[pl.ds(base, CH)])
            for i in range(CH // 16):
                sl = pl.ds(i * 16, 16)
                valid = (base + i * 16 + iota) < seg
                cbuf[sl] = jnp.where(valid, 1.0, 0.0)
            pltpu.sync_copy(cbuf, pval_ref.at[c, s, 0].at[pl.ds(base, CH)])
            return 0
        nch = lax.div(seg + (CH - 1), CH)
        lax.fori_loop(0, nch, chunk, 0)
        for i in range(CH // 16):
            ibuf[pl.ds(i * 16, 16)] = jnp.zeros((16,), jnp.int32)
            cbuf[pl.ds(i * 16, 16)] = jnp.zeros((16,), jnp.float32)

        def padchunk(kk, _):
            base = kk * CH
            pltpu.sync_copy(ibuf, psrc_ref.at[c, s, 0].at[pl.ds(base, CH)])
            pltpu.sync_copy(ibuf, pdst_ref.at[c, s, 0].at[pl.ds(base, CH)])
            pltpu.sync_copy(cbuf, pcoef_ref.at[c, s, 0].at[pl.ds(base, CH)])
            pltpu.sync_copy(cbuf, pval_ref.at[c, s, 0].at[pl.ds(base, CH)])
            return 0
        lax.fori_loop(nch, CAPC, padchunk, 0)

    return k(src_all, dst_all, coeff, permp, seglen)


def _sc_prop(feat2d, pk, nchunks):
    mesh = plsc.VectorSubcoreMesh(**_MESH)

    @functools.partial(
        pl.kernel,
        out_type=jax.ShapeDtypeStruct((2, NT, 1, RPT * F), jnp.float32),
        mesh=mesh,
        compiler_params=pltpu.CompilerParams(needs_layout_passes=False),
        scratch_types=[
            pltpu.VMEM((3 * CH,), jnp.int32),
            pltpu.VMEM((3 * CH,), jnp.int32),
            pltpu.VMEM((CH, F), jnp.float32),
            pltpu.VMEM((CH, F), jnp.float32),
            pltpu.VMEM((RPT * F,), jnp.float32),
            pltpu.VMEM((1, 16), jnp.int32),
            pltpu.SemaphoreType.DMA,
            pltpu.SemaphoreType.DMA,
        ],
    )
    def k(feat_ref, pk_ref, nch_ref, out_ref,
          pka, pkb, rowsa, rowsb, acc, lbuf, sema, semb):
        c = lax.axis_index("c")
        s = lax.axis_index("s")
        pltpu.sync_copy(nch_ref.at[c], lbuf)
        nch = plsc.load_gather(
            lbuf, [jnp.zeros((16,), jnp.int32),
                   jnp.full((16,), s, jnp.int32)])[0]

        def zr(r, _):
            for j in range(F // 16):
                acc[pl.ds(r * F + j * 16, 16)] = jnp.zeros((16,), jnp.float32)
            return 0
        lax.fori_loop(0, RPT, zr, 0)

        def load_fire(kk, pkx, rowsx, semx):
            pltpu.sync_copy(pk_ref.at[c, s, 0].at[pl.ds(kk * (3 * CH),
                                                        3 * CH)], pkx)
            pltpu.async_copy(feat_ref.at[pkx.at[pl.ds(0, CH)]], rowsx, semx)

        def waitx(rowsx, semx):
            pltpu.make_async_copy(feat_ref.at[pl.ds(0, CH), :], rowsx,
                                  semx).wait()

        def process(pkx, rowsx):
            def grp(i, _):
                dvec = pkx[pl.ds(CH + i * 16, 16)] * F
                cvec = lax.bitcast_convert_type(
                    pkx[pl.ds(2 * CH + i * 16, 16)], jnp.float32)
                for e in range(16):
                    cc = cvec[e]
                    db = dvec[e]
                    r = i * 16 + e
                    for j in range(F // 16):
                        fs = pl.ds(db + j * 16, 16)
                        acc[fs] = acc[fs] + rowsx[r, pl.ds(j * 16, 16)] * cc
                return 0
            lax.fori_loop(0, CH // 16, grp, 0)

        load_fire(0, pka, rowsa, sema)

        def pair(kk2, _):
            k0 = 2 * kk2
            k1 = k0 + 1

            @pl.when(k1 < nch)
            def _():
                load_fire(k1, pkb, rowsb, semb)
            waitx(rowsa, sema)
            process(pka, rowsa)

            @pl.when(k1 + 1 < nch)
            def _():
                load_fire(k1 + 1, pka, rowsa, sema)

            @pl.when(k1 < nch)
            def _():
                waitx(rowsb, semb)
                process(pkb, rowsb)
            return 0
        lax.fori_loop(0, (nch + 1) // 2, pair, 0)
        pltpu.sync_copy(acc, out_ref.at[c, s, 0])

    return k(feat2d, pk, nchunks)


def _sc_gat(featx, adfull, psrc, pdst, pval, nchunks):
    mesh = plsc.VectorSubcoreMesh(**_MESH)
    CHG = 64

    @functools.partial(
        pl.kernel,
        out_type=(jax.ShapeDtypeStruct((2, NPAD, F), jnp.float32),
                  jax.ShapeDtypeStruct((2, NT, 1, RPT * 16), jnp.float32)),
        mesh=mesh,
        compiler_params=pltpu.CompilerParams(needs_layout_passes=False),
        scratch_types=[
            pltpu.VMEM((CHG,), jnp.int32),
            pltpu.VMEM((CHG,), jnp.int32),
            pltpu.VMEM((CHG,), jnp.float32),
            pltpu.VMEM((CHG,), jnp.int32),
            pltpu.VMEM((CHG, F + 128), jnp.float32),
            pltpu.VMEM((CHG, 128), jnp.float32),
            pltpu.VMEM((RPT, F), jnp.float32),
            pltpu.VMEM((RPT * 16,), jnp.float32),
            pltpu.VMEM((1, 16), jnp.int32),
            pltpu.SemaphoreType.DMA,
            pltpu.SemaphoreType.DMA,
        ],
    )
    def k(feat_ref, ad_ref, src_ref, dst_ref, val_ref, nch_ref,
          out_ref, z_ref,
          sbuf, dbuf, vbuf, gbuf, rows, adrows, acc, accz, lbuf, sem, sem2):
        c = lax.axis_index("c")
        s = lax.axis_index("s")
        pltpu.sync_copy(nch_ref.at[c], lbuf)
        nch = plsc.load_gather(
            lbuf, [jnp.zeros((16,), jnp.int32),
                   jnp.full((16,), s, jnp.int32)])[0]

        def zr(r, _):
            for j in range(F // 16):
                acc[r, pl.ds(j * 16, 16)] = jnp.zeros((16,), jnp.float32)
            accz[pl.ds(r * 16, 16)] = jnp.zeros((16,), jnp.float32)
            return 0
        lax.fori_loop(0, RPT, zr, 0)

        goff = c * NPAD + s * RPT

        def chunk(kk, _):
            base = kk * CHG
            pltpu.sync_copy(src_ref.at[c, s, 0].at[pl.ds(base, CHG)], sbuf)
            pltpu.sync_copy(dst_ref.at[c, s, 0].at[pl.ds(base, CHG)], dbuf)
            pltpu.sync_copy(val_ref.at[c, s, 0].at[pl.ds(base, CHG)], vbuf)
            for i in range(CHG // 16):
                sl = pl.ds(i * 16, 16)
                gbuf[sl] = dbuf[sl] + goff
            cp1 = pltpu.async_copy(feat_ref.at[sbuf], rows, sem)
            cp2 = pltpu.async_copy(ad_ref.at[gbuf], adrows, sem2)
            cp1.wait()
            cp2.wait()

            def grp(i, _):
                sl = pl.ds(i * 16, 16)
                dvec = dbuf[sl]
                vvec = vbuf[sl]
                for e in range(16):
                    dl = dvec[e]
                    r = i * 16 + e
                    x = rows[r, pl.ds(F, 16)] + adrows[r, pl.ds(0, 16)]
                    x = jnp.maximum(x, x * 0.2)
                    pv = jnp.exp(x) * vvec[e]
                    for h in range(NHEAD):
                        ph = pv[h]
                        for j in range(F // (16 * NHEAD)):
                            fs = pl.ds(h * (F // NHEAD) + j * 16, 16)
                            acc[dl, fs] = acc[dl, fs] + rows[r, fs] * ph
                    accz[pl.ds(dl * 16, 16)] = accz[pl.ds(dl * 16, 16)] + pv
                return 0
            lax.fori_loop(0, CHG // 16, grp, 0)
            return 0
        lax.fori_loop(0, nch, chunk, 0)
        pltpu.sync_copy(acc, out_ref.at[c, pl.ds(s * RPT, RPT), :])
        pltpu.sync_copy(accz, z_ref.at[c, s, 0])

    return k(featx, adfull, psrc, pdst, pval, nchunks)


BM = 512
GB = NPAD // BM


def _tc_mm(x, w):
    ki, ko = w.shape[1], w.shape[2]

    def body(x_ref, w_ref, o_ref):
        o_ref[...] = jnp.dot(x_ref[0], w_ref[0],
                             preferred_element_type=jnp.float32)[None]

    return pl.pallas_call(
        body,
        grid=(2, GB),
        in_specs=[pl.BlockSpec((1, BM, ki), lambda g, i: (g, i, 0)),
                  pl.BlockSpec((1, ki, ko), lambda g, i: (g, 0, 0))],
        out_specs=pl.BlockSpec((1, BM, ko), lambda g, i: (g, i, 0)),
        out_shape=jax.ShapeDtypeStruct((2, NPAD, ko), jnp.float32),
    )(x, w)


def _tc_lrelu_mm(acc, b, w):
    ko = w.shape[2]

    def body(a_ref, b_ref, w_ref, x_ref, xw_ref):
        X = _lr(a_ref[0] + b_ref[0])
        x_ref[...] = X[None]
        xw_ref[...] = jnp.dot(X, w_ref[0],
                              preferred_element_type=jnp.float32)[None]

    return pl.pallas_call(
        body,
        grid=(2, GB),
        in_specs=[pl.BlockSpec((1, BM, F), lambda g, i: (g, i, 0)),
                  pl.BlockSpec((1, 1, F), lambda g, i: (g, 0, 0)),
                  pl.BlockSpec((1, F, ko), lambda g, i: (g, 0, 0))],
        out_specs=[pl.BlockSpec((1, BM, F), lambda g, i: (g, i, 0)),
                   pl.BlockSpec((1, BM, ko), lambda g, i: (g, i, 0))],
        out_shape=[jax.ShapeDtypeStruct((2, NPAD, F), jnp.float32),
                   jax.ShapeDtypeStruct((2, NPAD, ko), jnp.float32)],
    )(acc, b, w)


def _tc_gat_post(acc, zz, bg, w2, rmat):
    def body(a_ref, z_ref, bg_ref, w_ref, r_ref, x_ref, xw_ref):
        zr = jnp.dot(z_ref[0], r_ref[...], preferred_element_type=jnp.float32)
        X = _lr(a_ref[0] / jnp.maximum(zr, 1e-16) + bg_ref[0])
        x_ref[...] = X[None]
        xw_ref[...] = jnp.dot(X, w_ref[0],
                              preferred_element_type=jnp.float32)[None]

    return pl.pallas_call(
        body,
        grid=(2, GB),
        in_specs=[pl.BlockSpec((1, BM, F), lambda g, i: (g, i, 0)),
                  pl.BlockSpec((1, BM, 16), lambda g, i: (g, i, 0)),
                  pl.BlockSpec((1, 1, F), lambda g, i: (g, 0, 0)),
                  pl.BlockSpec((1, F, F), lambda g, i: (g, 0, 0)),
                  pl.BlockSpec((16, F), lambda g, i: (0, 0))],
        out_specs=[pl.BlockSpec((1, BM, F), lambda g, i: (g, i, 0)),
                   pl.BlockSpec((1, BM, F), lambda g, i: (g, i, 0))],
        out_shape=[jax.ShapeDtypeStruct((2, NPAD, F), jnp.float32),
                   jax.ShapeDtypeStruct((2, NPAD, F), jnp.float32)],
    )(acc, zz, bg, w2, rmat)


def _tc_mlp(acc3, b3, x1g, x2, betas, l1, c1, l2, c2, l3, c3):
    def body(a_ref, b_ref, x1_ref, x2_ref, bt_ref,
             l1_ref, c1_ref, l2_ref, c2_ref, l3_ref, c3_ref, o_ref):
        g = pl.program_id(0)
        b0 = bt_ref[g, 0]
        b1 = bt_ref[g, 1]
        X3 = _lr(a_ref[0] + b_ref[0])
        X = b0 * x1_ref[0] + b1 * x2_ref[0] + (1.0 - b0 - b1) * X3
        h = _lr(jnp.dot(X, l1_ref[0], preferred_element_type=jnp.float32)
                + c1_ref[0])
        h = _lr(jnp.dot(h, l2_ref[0], preferred_element_type=jnp.float32)
                + c2_ref[0])
        h = _lr(jnp.dot(h, l3_ref[0], preferred_element_type=jnp.float32)
                + c3_ref[0])
        o_ref[...] = h[None]

    return pl.pallas_call(
        body,
        grid=(2, GB),
        in_specs=[pl.BlockSpec((1, BM, F), lambda g, i: (g, i, 0)),
                  pl.BlockSpec((1, 1, F), lambda g, i: (g, 0, 0)),
                  pl.BlockSpec((1, BM, F), lambda g, i: (g, i, 0)),
                  pl.BlockSpec((1, BM, F), lambda g, i: (g, i, 0)),
                  pl.BlockSpec(memory_space=pltpu.SMEM),
                  pl.BlockSpec((1, F, F), lambda g, i: (g, 0, 0)),
                  pl.BlockSpec((1, 1, F), lambda g, i: (g, 0, 0)),
                  pl.BlockSpec((1, F, 128), lambda g, i: (g, 0, 0)),
                  pl.BlockSpec((1, 1, 128), lambda g, i: (g, 0, 0)),
                  pl.BlockSpec((1, 128, 64), lambda g, i: (g, 0, 0)),
                  pl.BlockSpec((1, 1, 64), lambda g, i: (g, 0, 0))],
        out_specs=pl.BlockSpec((1, BM, 64), lambda g, i: (g, i, 0)),
        out_shape=jax.ShapeDtypeStruct((2, NPAD, 64), jnp.float32),
    )(acc3, b3, x1g, x2, betas, l1, c1, l2, c2, l3, c3)


def _tc_final(feats):
    def body(x_ref, y_ref, o_ref):
        o_ref[...] = lax.dot_general(
            x_ref[0], y_ref[0], (((1,), (1,)), ((), ())),
            preferred_element_type=jnp.float32)

    return pl.pallas_call(
        body,
        grid=(GB, GB),
        in_specs=[pl.BlockSpec((1, BM, 64), lambda i, j: (0, i, 0)),
                  pl.BlockSpec((1, BM, 64), lambda i, j: (1, j, 0))],
        out_specs=pl.BlockSpec((BM, BM), lambda i, j: (i, j)),
        out_shape=jax.ShapeDtypeStruct((NPAD, NPAD), jnp.float32),
    )(feats, feats)


def kernel(edge_index_m, edge_index_d, data_m, data_d, x_m, x_d, beta1, beta2,
           gcn_x1_W, gcn_x1_b, gcn_x2_W, gcn_x2_b, gcn_x3_W, gcn_x3_b,
           gcn_y1_W, gcn_y1_b, gcn_y2_W, gcn_y2_b, gcn_y3_W, gcn_y3_b,
           gat_x_W, gat_x_as, gat_x_ad, gat_x_b,
           gat_y_W, gat_y_as, gat_y_ad, gat_y_b,
           lin_x1_W, lin_x1_b, lin_x2_W, lin_x2_b, lin_x3_W, lin_x3_b,
           lin_y1_W, lin_y1_b, lin_y2_W, lin_y2_b, lin_y3_W, lin_y3_b):
    ei_m = edge_index_m.astype(jnp.int32)
    ei_d = edge_index_d.astype(jnp.int32)

    selfn = jnp.arange(NPAD, dtype=jnp.int32)
    src = jnp.stack([jnp.concatenate([ei_m[0], selfn]),
                     jnp.concatenate([ei_d[0], selfn])])
    dst = jnp.stack([jnp.concatenate([ei_m[1], selfn]),
                     jnp.concatenate([ei_d[1], selfn])])
    flat = jnp.stack([ei_m[0] * NN + ei_m[1], ei_d[0] * NN + ei_d[1]])

    flat_k1 = flat.reshape(2, NT, K1C, CH)
    src_k1 = src[:, :NE].reshape(2, NT, K1C, CH)
    dst_k1 = dst[:, :NE].reshape(2, NT, K1C, CH)

    buck = dst // RPT
    perm = jnp.argsort(buck, axis=1).astype(jnp.int32)
    bsort = jnp.take_along_axis(buck, perm, axis=1)
    tt = jnp.arange(NT, dtype=jnp.int32)
    starts = jax.vmap(lambda bs: jnp.searchsorted(bs, tt, side="left")
                      )(bsort).astype(jnp.int32)
    ends = jax.vmap(lambda bs: jnp.searchsorted(bs, tt, side="right")
                    )(bsort).astype(jnp.int32)
    seglen = ends - starts
    pos = starts[:, :, None] + jnp.arange(CAP, dtype=jnp.int32)[None, None, :]
    pos = jnp.minimum(pos, EP - 1)
    permp = jnp.take_along_axis(perm, pos.reshape(2, -1), axis=1
                                ).reshape(2, NT, CAP)
    permp = permp + (jnp.arange(2, dtype=jnp.int32) * EP)[:, None, None]
    permp = permp[:, :, None, :]
    nchunks = ((seglen + (CH - 1)) // CH)[:, None, :]
    nchunks64 = ((seglen + 63) // 64)[:, None, :]
    seglen = seglen[:, None, :]

    coeff = _sc_prep(data_m.reshape(-1), data_d.reshape(-1),
                     flat_k1, src_k1, dst_k1)
    psrc, pdst, pcoef, pval = _sc_permute(
        src.reshape(-1), dst.reshape(-1), coeff.reshape(-1), permp, seglen)

    pad = ((0, NPAD - NN), (0, 0))
    xpad = jnp.stack([jnp.pad(x_m, pad), jnp.pad(x_d, pad)])

    w1 = jnp.stack([gcn_x1_W, gcn_y1_W])
    b1 = jnp.stack([gcn_x1_b, gcn_y1_b])[:, None, :]
    w2 = jnp.stack([gcn_x2_W, gcn_y2_W])
    b2 = jnp.stack([gcn_x2_b, gcn_y2_b])[:, None, :]
    w3 = jnp.stack([gcn_x3_W, gcn_y3_W])
    b3 = jnp.stack([gcn_x3_b, gcn_y3_b])[:, None, :]
    wg = jnp.stack([gat_x_W, gat_y_W])
    bg = jnp.stack([gat_x_b, gat_y_b])[:, None, :]

    rep = jnp.repeat(jnp.eye(NHEAD, dtype=jnp.float32), F // NHEAD, axis=0)
    amat_s = jnp.stack([gat_x_as.reshape(-1)[:, None] * rep,
                        gat_y_as.reshape(-1)[:, None] * rep])
    amat_d = jnp.stack([gat_x_ad.reshape(-1)[:, None] * rep,
                        gat_y_ad.reshape(-1)[:, None] * rep])
    rmat = jnp.concatenate(
        [jnp.repeat(jnp.eye(NHEAD, dtype=jnp.float32), F // NHEAD, axis=1),
         jnp.zeros((16 - NHEAD, F), jnp.float32)], axis=0)

    pk = jnp.stack(
        [psrc.reshape(2, NT, 1, CAP // CH, CH),
         pdst.reshape(2, NT, 1, CAP // CH, CH),
         lax.bitcast_convert_type(pcoef, jnp.int32
                                  ).reshape(2, NT, 1, CAP // CH, CH)],
        axis=4).reshape(2, NT, 1, 3 * CAP)

    xw1 = _tc_mm(xpad, w1)
    acc1 = _sc_prop(xw1.reshape(2 * NPAD, F), pk, nchunks
                    ).reshape(2, NPAD, F)
    _, xwg = _tc_lrelu_mm(acc1, b1, wg)
    as_t = _tc_mm(xwg, amat_s)
    ad_t = _tc_mm(xwg, amat_d)
    featx = jnp.concatenate(
        [xwg, as_t, jnp.zeros((2, NPAD, 124), jnp.float32)],
        axis=2).reshape(2 * NPAD, F + 128)
    adfull = jnp.concatenate(
        [ad_t, jnp.zeros((2, NPAD, 124), jnp.float32)],
        axis=2).reshape(2 * NPAD, 128)
    accg, zg = _sc_gat(featx, adfull, psrc, pdst, pval, nchunks64)
    zg = zg.reshape(2, NPAD, 16)
    x1g, xw2 = _tc_gat_post(accg, zg, bg, w2, rmat)
    acc2 = _sc_prop(xw2.reshape(2 * NPAD, F), pk, nchunks
                    ).reshape(2, NPAD, F)
    x2, xw3 = _tc_lrelu_mm(acc2, b2, w3)
    acc3 = _sc_prop(xw3.reshape(2 * NPAD, F), pk, nchunks
                    ).reshape(2, NPAD, F)

    betas = jnp.stack([beta1, beta2])
    l1 = jnp.stack([lin_x1_W, lin_y1_W])
    c1 = jnp.stack([lin_x1_b, lin_y1_b])[:, None, :]
    l2 = jnp.stack([lin_x2_W, lin_y2_W])
    c2 = jnp.stack([lin_x2_b, lin_y2_b])[:, None, :]
    l3 = jnp.stack([lin_x3_W, lin_y3_W])
    c3 = jnp.stack([lin_x3_b, lin_y3_b])[:, None, :]
    feats = _tc_mlp(acc3, b3, x1g, x2, betas, l1, c1, l2, c2, l3, c3)
    return _tc_final(feats)[:NN, :NN]


# speedup vs baseline: 6.1893x; 1.0108x over previous
import functools

import jax
import jax.numpy as jnp
from jax import lax
from jax.experimental import pallas as pl
from jax.experimental.pallas import tpu as pltpu
from jax.experimental.pallas import tpu_sc as plsc

NN = 5000
NPAD = 5120
NE = 160000
EP = NE + NPAD
NT = 16
CH = 80
K1C = (NE // NT) // CH
RPT = NPAD // NT
F = 256
NHEAD = 4
FZ = F + 16
CAP = 16000
CAPC = CAP // CH

_MESH = dict(core_axis_name="c", subcore_axis_name="s")


def _lr(v):
    return jnp.where(v > 0, v, v * 0.01)


def _qrsqrt(x):
    i = lax.bitcast_convert_type(x, jnp.int32)
    i = jnp.int32(0x5F3759DF) - lax.shift_right_logical(i, 1)
    y = lax.bitcast_convert_type(i, jnp.float32)
    for _ in range(3):
        y = y * (1.5 - 0.5 * x * y * y)
    return y


def _sc_prep(data_m_flat, data_d_flat, flat_k1, src_k1, dst_k1):
    mesh = plsc.VectorSubcoreMesh(**_MESH)

    @functools.partial(
        pl.kernel,
        out_type=jax.ShapeDtypeStruct((2, 1, EP), jnp.float32),
        mesh=mesh,
        compiler_params=pltpu.CompilerParams(needs_layout_passes=False),
        scratch_types=[
            pltpu.VMEM((K1C, CH), jnp.int32),
            pltpu.VMEM((K1C, CH), jnp.int32),
            pltpu.VMEM((K1C, CH), jnp.float32),
            pltpu.VMEM((RPT,), jnp.float32),
            pltpu.VMEM((NPAD,), jnp.float32),
            pltpu.VMEM_SHARED((NPAD,), jnp.float32),
            pltpu.SemaphoreType.DMA,
        ],
    )
    def k(dm_ref, dd_ref, flat_ref, src_ref, dst_ref, coef_ref,
          fbuf, dbuf, wbuf, dv, dinvtab, deg_sh, sem):
        c = lax.axis_index("c")
        s = lax.axis_index("s")
        pltpu.sync_copy(flat_ref.at[c, s], fbuf)
        pltpu.sync_copy(dst_ref.at[c, s], dbuf)
        for j in range(RPT // 16):
            dv[pl.ds(j * 16, 16)] = jnp.full((16,), 1.0, jnp.float32)
        pltpu.sync_copy(dv, deg_sh.at[pl.ds(s * RPT, RPT)])
        plsc.subcore_barrier()

        def gather_all(dref):
            def chunk(kk, _):
                pltpu.async_copy(dref.at[fbuf.at[kk]], wbuf.at[kk], sem).wait()
                return 0
            lax.fori_loop(0, K1C, chunk, 0)

        @pl.when(c == 0)
        def _():
            gather_all(dm_ref)

        @pl.when(c == 1)
        def _():
            gather_all(dd_ref)

        def degadd(kk, _):
            pltpu.sync_copy(wbuf.at[kk], deg_sh.at[dbuf.at[kk]], add=True)
            return 0
        lax.fori_loop(0, K1C, degadd, 0)
        plsc.subcore_barrier()

        pltpu.sync_copy(deg_sh.at[pl.ds(s * RPT, RPT)], dv)
        for j in range(RPT // 16):
            dv[pl.ds(j * 16, 16)] = _qrsqrt(dv[pl.ds(j * 16, 16)])
        pltpu.sync_copy(dv, deg_sh.at[pl.ds(s * RPT, RPT)])
        for j in range(RPT // 16):
            y = dv[pl.ds(j * 16, 16)]
            dv[pl.ds(j * 16, 16)] = y * y
        pltpu.sync_copy(dv, coef_ref.at[c, 0].at[pl.ds(NE + s * RPT, RPT)])
        plsc.subcore_barrier()

        pltpu.sync_copy(deg_sh, dinvtab)
        pltpu.sync_copy(src_ref.at[c, s], fbuf)

        def normchunk(kk, _):
            def sub(i, _):
                sl = pl.ds(i * 16, 16)
                sv = fbuf[kk, sl]
                tv = dbuf[kk, sl]
                dsv = plsc.load_gather(dinvtab, [sv])
                dtv = plsc.load_gather(dinvtab, [tv])
                wbuf[kk, sl] = dsv * wbuf[kk, sl] * dtv
                return 0
            lax.fori_loop(0, CH // 16, sub, 0)
            pltpu.sync_copy(
                wbuf.at[kk],
                coef_ref.at[c, 0].at[pl.ds(s * (K1C * CH) + kk * CH, CH)])
            return 0
        lax.fori_loop(0, K1C, normchunk, 0)

    return k(data_m_flat, data_d_flat, flat_k1, src_k1, dst_k1)


def _sc_permute(src_all, dst_all, coeff, permp, seglen):
    mesh = plsc.VectorSubcoreMesh(**_MESH)

    @functools.partial(
        pl.kernel,
        out_type=(
            jax.ShapeDtypeStruct((2, NT, 1, CAP), jnp.int32),
            jax.ShapeDtypeStruct((2, NT, 1, CAP), jnp.int32),
            jax.ShapeDtypeStruct((2, NT, 1, CAP), jnp.float32),
            jax.ShapeDtypeStruct((2, NT, 1, CAP), jnp.float32),
        ),
        mesh=mesh,
        compiler_params=pltpu.CompilerParams(needs_layout_passes=False),
        scratch_types=[
            pltpu.VMEM((CH,), jnp.int32),
            pltpu.VMEM((CH,), jnp.int32),
            pltpu.VMEM((CH,), jnp.float32),
            pltpu.VMEM((1, 16), jnp.int32),
            pltpu.SemaphoreType.DMA,
        ],
    )
    def k(src_ref, dst_ref, coef_ref, perm_ref, len_ref,
          psrc_ref, pdst_ref, pcoef_ref, pval_ref, pbuf, ibuf, cbuf, lbuf,
          sem):
        c = lax.axis_index("c")
        s = lax.axis_index("s")
        pltpu.sync_copy(len_ref.at[c], lbuf)
        seg = plsc.load_gather(
            lbuf, [jnp.zeros((16,), jnp.int32),
                   jnp.full((16,), s, jnp.int32)])[0]
        iota = lax.iota(jnp.int32, 16)

        def chunk(kk, _):
            pltpu.sync_copy(perm_ref.at[c, s, 0].at[pl.ds(kk * CH, CH)], pbuf)
            base = kk * CH
            pltpu.async_copy(src_ref.at[pbuf], ibuf, sem).wait()
            for i in range(CH // 16):
                sl = pl.ds(i * 16, 16)
                ibuf[sl] = ibuf[sl] + c * NPAD
            pltpu.sync_copy(ibuf, psrc_ref.at[c, s, 0].at[pl.ds(base, CH)])
            pltpu.async_copy(dst_ref.at[pbuf], ibuf, sem).wait()
            for i in range(CH // 16):
                sl = pl.ds(i * 16, 16)
                valid = (base + i * 16 + iota) < seg
                ibuf[sl] = jnp.where(valid, ibuf[sl] - s * RPT, 0)
            pltpu.sync_copy(ibuf, pdst_ref.at[c, s, 0].at[pl.ds(base, CH)])
            pltpu.async_copy(coef_ref.at[pbuf], cbuf, sem).wait()
            for i in range(CH // 16):
                sl = pl.ds(i * 16, 16)
                valid = (base + i * 16 + iota) < seg
                cbuf[sl] = jnp.where(valid, cbuf[sl], 0.0)
            pltpu.sync_copy(cbuf, pcoef_ref.at[c, s, 0].at[pl.ds(base, CH)])
            for i in range(CH // 16):
                sl = pl.ds(i * 16, 16)
                valid = (base + i * 16 + iota) < seg
                cbuf[sl] = jnp.where(valid, 1.0, 0.0)
            pltpu.sync_copy(cbuf, pval_ref.at[c, s, 0].at[pl.ds(base, CH)])
            return 0
        nch = lax.div(seg + (CH - 1), CH)
        lax.fori_loop(0, nch, chunk, 0)
        for i in range(CH // 16):
            ibuf[pl.ds(i * 16, 16)] = jnp.zeros((16,), jnp.int32)
            cbuf[pl.ds(i * 16, 16)] = jnp.zeros((16,), jnp.float32)

        def padchunk(kk, _):
            base = kk * CH
            pltpu.sync_copy(ibuf, psrc_ref.at[c, s, 0].at[pl.ds(base, CH)])
            pltpu.sync_copy(ibuf, pdst_ref.at[c, s, 0].at[pl.ds(base, CH)])
            pltpu.sync_copy(cbuf, pcoef_ref.at[c, s, 0].at[pl.ds(base, CH)])
            pltpu.sync_copy(cbuf, pval_ref.at[c, s, 0].at[pl.ds(base, CH)])
            return 0
        lax.fori_loop(nch, CAPC, padchunk, 0)

    return k(src_all, dst_all, coeff, permp, seglen)


def _sc_prop(feat2d, pk, nchunks):
    mesh = plsc.VectorSubcoreMesh(**_MESH)

    @functools.partial(
        pl.kernel,
        out_type=jax.ShapeDtypeStruct((2, NT, 1, RPT * F), jnp.float32),
        mesh=mesh,
        compiler_params=pltpu.CompilerParams(needs_layout_passes=False),
        scratch_types=[
            pltpu.VMEM((3 * CH,), jnp.int32),
            pltpu.VMEM((3 * CH,), jnp.int32),
            pltpu.VMEM((CH, F), jnp.float32),
            pltpu.VMEM((CH, F), jnp.float32),
            pltpu.VMEM((RPT * F,), jnp.float32),
            pltpu.VMEM((1, 16), jnp.int32),
            pltpu.SemaphoreType.DMA,
            pltpu.SemaphoreType.DMA,
        ],
    )
    def k(feat_ref, pk_ref, nch_ref, out_ref,
          pka, pkb, rowsa, rowsb, acc, lbuf, sema, semb):
        c = lax.axis_index("c")
        s = lax.axis_index("s")
        pltpu.sync_copy(nch_ref.at[c], lbuf)
        nch = plsc.load_gather(
            lbuf, [jnp.zeros((16,), jnp.int32),
                   jnp.full((16,), s, jnp.int32)])[0]

        def zr(r, _):
            for j in range(F // 16):
                acc[pl.ds(r * F + j * 16, 16)] = jnp.zeros((16,), jnp.float32)
            return 0
        lax.fori_loop(0, RPT, zr, 0)

        def load_fire(kk, pkx, rowsx, semx):
            pltpu.sync_copy(pk_ref.at[c, s, 0].at[pl.ds(kk * (3 * CH),
                                                        3 * CH)], pkx)
            pltpu.async_copy(feat_ref.at[pkx.at[pl.ds(0, CH)]], rowsx, semx)

        def waitx(rowsx, semx):
            pltpu.make_async_copy(feat_ref.at[pl.ds(0, CH), :], rowsx,
                                  semx).wait()

        def process(pkx, rowsx):
            def grp(i, _):
                dvec = pkx[pl.ds(CH + i * 16, 16)] * F
                cvec = lax.bitcast_convert_type(
                    pkx[pl.ds(2 * CH + i * 16, 16)], jnp.float32)
                for e in range(16):
                    cc = cvec[e]
                    db = dvec[e]
                    r = i * 16 + e
                    for j in range(F // 16):
                        fs = pl.ds(db + j * 16, 16)
                        acc[fs] = acc[fs] + rowsx[r, pl.ds(j * 16, 16)] * cc
                return 0
            lax.fori_loop(0, CH // 16, grp, 0)

        load_fire(0, pka, rowsa, sema)

        def pair(kk2, _):
            k0 = 2 * kk2
            k1 = k0 + 1

            @pl.when(k1 < nch)
            def _():
                load_fire(k1, pkb, rowsb, semb)
            waitx(rowsa, sema)
            process(pka, rowsa)

            @pl.when(k1 + 1 < nch)
            def _():
                load_fire(k1 + 1, pka, rowsa, sema)

            @pl.when(k1 < nch)
            def _():
                waitx(rowsb, semb)
                process(pkb, rowsb)
            return 0
        lax.fori_loop(0, (nch + 1) // 2, pair, 0)
        pltpu.sync_copy(acc, out_ref.at[c, s, 0])

    return k(feat2d, pk, nchunks)


def _sc_gat(featx, adfull, pkg_arr, nchunks):
    mesh = plsc.VectorSubcoreMesh(**_MESH)
    CHG = 64

    @functools.partial(
        pl.kernel,
        out_type=(jax.ShapeDtypeStruct((2, NPAD, F), jnp.float32),
                  jax.ShapeDtypeStruct((2, NT, 1, RPT * 16), jnp.float32)),
        mesh=mesh,
        compiler_params=pltpu.CompilerParams(needs_layout_passes=False),
        scratch_types=[
            pltpu.VMEM((3 * CHG,), jnp.int32),
            pltpu.VMEM((CHG,), jnp.int32),
            pltpu.VMEM((CHG, F + 128), jnp.float32),
            pltpu.VMEM((CHG, 128), jnp.float32),
            pltpu.VMEM((RPT, F), jnp.float32),
            pltpu.VMEM((RPT * 16,), jnp.float32),
            pltpu.VMEM((1, 16), jnp.int32),
            pltpu.SemaphoreType.DMA,
            pltpu.SemaphoreType.DMA,
        ],
    )
    def k(feat_ref, ad_ref, pk_ref, nch_ref,
          out_ref, z_ref,
          pkg, gbuf, rows, adrows, acc, accz, lbuf, sem, sem2):
        c = lax.axis_index("c")
        s = lax.axis_index("s")
        pltpu.sync_copy(nch_ref.at[c], lbuf)
        nch = plsc.load_gather(
            lbuf, [jnp.zeros((16,), jnp.int32),
                   jnp.full((16,), s, jnp.int32)])[0]

        def zr(r, _):
            for j in range(F // 16):
                acc[r, pl.ds(j * 16, 16)] = jnp.zeros((16,), jnp.float32)
            accz[pl.ds(r * 16, 16)] = jnp.zeros((16,), jnp.float32)
            return 0
        lax.fori_loop(0, RPT, zr, 0)

        goff = c * NPAD + s * RPT

        def chunk(kk, _):
            pltpu.sync_copy(pk_ref.at[c, s, 0].at[pl.ds(kk * (3 * CHG),
                                                        3 * CHG)], pkg)
            for i in range(CHG // 16):
                sl = pl.ds(i * 16, 16)
                gbuf[sl] = pkg[pl.ds(CHG + i * 16, 16)] + goff
            cp1 = pltpu.async_copy(feat_ref.at[pkg.at[pl.ds(0, CHG)]], rows,
                                   sem)
            cp2 = pltpu.async_copy(ad_ref.at[gbuf], adrows, sem2)
            cp1.wait()
            cp2.wait()

            def grp(i, _):
                dvec = pkg[pl.ds(CHG + i * 16, 16)]
                vvec = lax.bitcast_convert_type(
                    pkg[pl.ds(2 * CHG + i * 16, 16)], jnp.float32)
                for e in range(16):
                    dl = dvec[e]
                    r = i * 16 + e
                    x = rows[r, pl.ds(F, 16)] + adrows[r, pl.ds(0, 16)]
                    x = jnp.maximum(x, x * 0.2)
                    pv = jnp.exp(x) * vvec[e]
                    for h in range(NHEAD):
                        ph = pv[h]
                        for j in range(F // (16 * NHEAD)):
                            fs = pl.ds(h * (F // NHEAD) + j * 16, 16)
                            acc[dl, fs] = acc[dl, fs] + rows[r, fs] * ph
                    accz[pl.ds(dl * 16, 16)] = accz[pl.ds(dl * 16, 16)] + pv
                return 0
            lax.fori_loop(0, CHG // 16, grp, 0)
            return 0
        lax.fori_loop(0, nch, chunk, 0)
        pltpu.sync_copy(acc, out_ref.at[c, pl.ds(s * RPT, RPT), :])
        pltpu.sync_copy(accz, z_ref.at[c, s, 0])

    return k(featx, adfull, pkg_arr, nchunks)


BM = 512
GB = NPAD // BM


def _tc_mm(x, w):
    ki, ko = w.shape[1], w.shape[2]

    def body(x_ref, w_ref, o_ref):
        o_ref[...] = jnp.dot(x_ref[0], w_ref[0],
                             preferred_element_type=jnp.float32)[None]

    return pl.pallas_call(
        body,
        grid=(2, GB),
        in_specs=[pl.BlockSpec((1, BM, ki), lambda g, i: (g, i, 0)),
                  pl.BlockSpec((1, ki, ko), lambda g, i: (g, 0, 0))],
        out_specs=pl.BlockSpec((1, BM, ko), lambda g, i: (g, i, 0)),
        out_shape=jax.ShapeDtypeStruct((2, NPAD, ko), jnp.float32),
    )(x, w)


def _tc_lrelu_mm(acc, b, w):
    ko = w.shape[2]

    def body(a_ref, b_ref, w_ref, x_ref, xw_ref):
        X = _lr(a_ref[0] + b_ref[0])
        x_ref[...] = X[None]
        xw_ref[...] = jnp.dot(X, w_ref[0],
                              preferred_element_type=jnp.float32)[None]

    return pl.pallas_call(
        body,
        grid=(2, GB),
        in_specs=[pl.BlockSpec((1, BM, F), lambda g, i: (g, i, 0)),
                  pl.BlockSpec((1, 1, F), lambda g, i: (g, 0, 0)),
                  pl.BlockSpec((1, F, ko), lambda g, i: (g, 0, 0))],
        out_specs=[pl.BlockSpec((1, BM, F), lambda g, i: (g, i, 0)),
                   pl.BlockSpec((1, BM, ko), lambda g, i: (g, i, 0))],
        out_shape=[jax.ShapeDtypeStruct((2, NPAD, F), jnp.float32),
                   jax.ShapeDtypeStruct((2, NPAD, ko), jnp.float32)],
    )(acc, b, w)


def _tc_gat_post(acc, zz, bg, w2, rmat):
    def body(a_ref, z_ref, bg_ref, w_ref, r_ref, x_ref, xw_ref):
        zr = jnp.dot(z_ref[0], r_ref[...], preferred_element_type=jnp.float32)
        X = _lr(a_ref[0] / jnp.maximum(zr, 1e-16) + bg_ref[0])
        x_ref[...] = X[None]
        xw_ref[...] = jnp.dot(X, w_ref[0],
                              preferred_element_type=jnp.float32)[None]

    return pl.pallas_call(
        body,
        grid=(2, GB),
        in_specs=[pl.BlockSpec((1, BM, F), lambda g, i: (g, i, 0)),
                  pl.BlockSpec((1, BM, 16), lambda g, i: (g, i, 0)),
                  pl.BlockSpec((1, 1, F), lambda g, i: (g, 0, 0)),
                  pl.BlockSpec((1, F, F), lambda g, i: (g, 0, 0)),
                  pl.BlockSpec((16, F), lambda g, i: (0, 0))],
        out_specs=[pl.BlockSpec((1, BM, F), lambda g, i: (g, i, 0)),
                   pl.BlockSpec((1, BM, F), lambda g, i: (g, i, 0))],
        out_shape=[jax.ShapeDtypeStruct((2, NPAD, F), jnp.float32),
                   jax.ShapeDtypeStruct((2, NPAD, F), jnp.float32)],
    )(acc, zz, bg, w2, rmat)


def _tc_mlp(acc3, b3, x1g, x2, betas, l1, c1, l2, c2, l3, c3):
    def body(a_ref, b_ref, x1_ref, x2_ref, bt_ref,
             l1_ref, c1_ref, l2_ref, c2_ref, l3_ref, c3_ref, o_ref):
        g = pl.program_id(0)
        b0 = bt_ref[g, 0]
        b1 = bt_ref[g, 1]
        X3 = _lr(a_ref[0] + b_ref[0])
        X = b0 * x1_ref[0] + b1 * x2_ref[0] + (1.0 - b0 - b1) * X3
        h = _lr(jnp.dot(X, l1_ref[0], preferred_element_type=jnp.float32)
                + c1_ref[0])
        h = _lr(jnp.dot(h, l2_ref[0], preferred_element_type=jnp.float32)
                + c2_ref[0])
        h = _lr(jnp.dot(h, l3_ref[0], preferred_element_type=jnp.float32)
                + c3_ref[0])
        o_ref[...] = h[None]

    return pl.pallas_call(
        body,
        grid=(2, GB),
        in_specs=[pl.BlockSpec((1, BM, F), lambda g, i: (g, i, 0)),
                  pl.BlockSpec((1, 1, F), lambda g, i: (g, 0, 0)),
                  pl.BlockSpec((1, BM, F), lambda g, i: (g, i, 0)),
                  pl.BlockSpec((1, BM, F), lambda g, i: (g, i, 0)),
                  pl.BlockSpec(memory_space=pltpu.SMEM),
                  pl.BlockSpec((1, F, F), lambda g, i: (g, 0, 0)),
                  pl.BlockSpec((1, 1, F), lambda g, i: (g, 0, 0)),
                  pl.BlockSpec((1, F, 128), lambda g, i: (g, 0, 0)),
                  pl.BlockSpec((1, 1, 128), lambda g, i: (g, 0, 0)),
                  pl.BlockSpec((1, 128, 64), lambda g, i: (g, 0, 0)),
                  pl.BlockSpec((1, 1, 64), lambda g, i: (g, 0, 0))],
        out_specs=pl.BlockSpec((1, BM, 64), lambda g, i: (g, i, 0)),
        out_shape=jax.ShapeDtypeStruct((2, NPAD, 64), jnp.float32),
    )(acc3, b3, x1g, x2, betas, l1, c1, l2, c2, l3, c3)


def _tc_final(feats):
    def body(x_ref, y_ref, o_ref):
        o_ref[...] = lax.dot_general(
            x_ref[0], y_ref[0], (((1,), (1,)), ((), ())),
            preferred_element_type=jnp.float32)

    return pl.pallas_call(
        body,
        grid=(GB, GB),
        in_specs=[pl.BlockSpec((1, BM, 64), lambda i, j: (0, i, 0)),
                  pl.BlockSpec((1, BM, 64), lambda i, j: (1, j, 0))],
        out_specs=pl.BlockSpec((BM, BM), lambda i, j: (i, j)),
        out_shape=jax.ShapeDtypeStruct((NPAD, NPAD), jnp.float32),
    )(feats, feats)


def kernel(edge_index_m, edge_index_d, data_m, data_d, x_m, x_d, beta1, beta2,
           gcn_x1_W, gcn_x1_b, gcn_x2_W, gcn_x2_b, gcn_x3_W, gcn_x3_b,
           gcn_y1_W, gcn_y1_b, gcn_y2_W, gcn_y2_b, gcn_y3_W, gcn_y3_b,
           gat_x_W, gat_x_as, gat_x_ad, gat_x_b,
           gat_y_W, gat_y_as, gat_y_ad, gat_y_b,
           lin_x1_W, lin_x1_b, lin_x2_W, lin_x2_b, lin_x3_W, lin_x3_b,
           lin_y1_W, lin_y1_b, lin_y2_W, lin_y2_b, lin_y3_W, lin_y3_b):
    ei_m = edge_index_m.astype(jnp.int32)
    ei_d = edge_index_d.astype(jnp.int32)

    selfn = jnp.arange(NPAD, dtype=jnp.int32)
    src = jnp.stack([jnp.concatenate([ei_m[0], selfn]),
                     jnp.concatenate([ei_d[0], selfn])])
    dst = jnp.stack([jnp.concatenate([ei_m[1], selfn]),
                     jnp.concatenate([ei_d[1], selfn])])
    flat = jnp.stack([ei_m[0] * NN + ei_m[1], ei_d[0] * NN + ei_d[1]])

    flat_k1 = flat.reshape(2, NT, K1C, CH)
    src_k1 = src[:, :NE].reshape(2, NT, K1C, CH)
    dst_k1 = dst[:, :NE].reshape(2, NT, K1C, CH)

    buck = dst // RPT
    perm = jnp.argsort(buck, axis=1).astype(jnp.int32)
    bsort = jnp.take_along_axis(buck, perm, axis=1)
    tt = jnp.arange(NT, dtype=jnp.int32)
    starts = jax.vmap(lambda bs: jnp.searchsorted(bs, tt, side="left")
                      )(bsort).astype(jnp.int32)
    ends = jax.vmap(lambda bs: jnp.searchsorted(bs, tt, side="right")
                    )(bsort).astype(jnp.int32)
    seglen = ends - starts
    pos = starts[:, :, None] + jnp.arange(CAP, dtype=jnp.int32)[None, None, :]
    pos = jnp.minimum(pos, EP - 1)
    permp = jnp.take_along_axis(perm, pos.reshape(2, -1), axis=1
                                ).reshape(2, NT, CAP)
    permp = permp + (jnp.arange(2, dtype=jnp.int32) * EP)[:, None, None]
    permp = permp[:, :, None, :]
    nchunks = ((seglen + (CH - 1)) // CH)[:, None, :]
    nchunks64 = ((seglen + 63) // 64)[:, None, :]
    seglen = seglen[:, None, :]

    coeff = _sc_prep(data_m.reshape(-1), data_d.reshape(-1),
                     flat_k1, src_k1, dst_k1)
    psrc, pdst, pcoef, pval = _sc_permute(
        src.reshape(-1), dst.reshape(-1), coeff.reshape(-1), permp, seglen)

    pad = ((0, NPAD - NN), (0, 0))
    xpad = jnp.stack([jnp.pad(x_m, pad), jnp.pad(x_d, pad)])

    w1 = jnp.stack([gcn_x1_W, gcn_y1_W])
    b1 = jnp.stack([gcn_x1_b, gcn_y1_b])[:, None, :]
    w2 = jnp.stack([gcn_x2_W, gcn_y2_W])
    b2 = jnp.stack([gcn_x2_b, gcn_y2_b])[:, None, :]
    w3 = jnp.stack([gcn_x3_W, gcn_y3_W])
    b3 = jnp.stack([gcn_x3_b, gcn_y3_b])[:, None, :]
    wg = jnp.stack([gat_x_W, gat_y_W])
    bg = jnp.stack([gat_x_b, gat_y_b])[:, None, :]

    rep = jnp.repeat(jnp.eye(NHEAD, dtype=jnp.float32), F // NHEAD, axis=0)
    amat_s = jnp.stack([gat_x_as.reshape(-1)[:, None] * rep,
                        gat_y_as.reshape(-1)[:, None] * rep])
    amat_d = jnp.stack([gat_x_ad.reshape(-1)[:, None] * rep,
                        gat_y_ad.reshape(-1)[:, None] * rep])
    rmat = jnp.concatenate(
        [jnp.repeat(jnp.eye(NHEAD, dtype=jnp.float32), F // NHEAD, axis=1),
         jnp.zeros((16 - NHEAD, F), jnp.float32)], axis=0)

    pk = jnp.stack(
        [psrc.reshape(2, NT, 1, CAP // CH, CH),
         pdst.reshape(2, NT, 1, CAP // CH, CH),
         lax.bitcast_convert_type(pcoef, jnp.int32
                                  ).reshape(2, NT, 1, CAP // CH, CH)],
        axis=4).reshape(2, NT, 1, 3 * CAP)

    xw1 = _tc_mm(xpad, w1)
    acc1 = _sc_prop(xw1.reshape(2 * NPAD, F), pk, nchunks
                    ).reshape(2, NPAD, F)
    _, xwg = _tc_lrelu_mm(acc1, b1, wg)
    as_t = _tc_mm(xwg, amat_s)
    ad_t = _tc_mm(xwg, amat_d)
    featx = jnp.concatenate(
        [xwg, as_t, jnp.zeros((2, NPAD, 124), jnp.float32)],
        axis=2).reshape(2 * NPAD, F + 128)
    adfull = jnp.concatenate(
        [ad_t, jnp.zeros((2, NPAD, 124), jnp.float32)],
        axis=2).reshape(2 * NPAD, 128)
    pkg = jnp.stack(
        [psrc.reshape(2, NT, 1, CAP // 64, 64),
         pdst.reshape(2, NT, 1, CAP // 64, 64),
         lax.bitcast_convert_type(pval, jnp.int32
                                  ).reshape(2, NT, 1, CAP // 64, 64)],
        axis=4).reshape(2, NT, 1, 3 * CAP)
    accg, zg = _sc_gat(featx, adfull, pkg, nchunks64)
    zg = zg.reshape(2, NPAD, 16)
    x1g, xw2 = _tc_gat_post(accg, zg, bg, w2, rmat)
    acc2 = _sc_prop(xw2.reshape(2 * NPAD, F), pk, nchunks
                    ).reshape(2, NPAD, F)
    x2, xw3 = _tc_lrelu_mm(acc2, b2, w3)
    acc3 = _sc_prop(xw3.reshape(2 * NPAD, F), pk, nchunks
                    ).reshape(2, NPAD, F)

    betas = jnp.stack([beta1, beta2])
    l1 = jnp.stack([lin_x1_W, lin_y1_W])
    c1 = jnp.stack([lin_x1_b, lin_y1_b])[:, None, :]
    l2 = jnp.stack([lin_x2_W, lin_y2_W])
    c2 = jnp.stack([lin_x2_b, lin_y2_b])[:, None, :]
    l3 = jnp.stack([lin_x3_W, lin_y3_W])
    c3 = jnp.stack([lin_x3_b, lin_y3_b])[:, None, :]
    feats = _tc_mlp(acc3, b3, x1g, x2, betas, l1, c1, l2, c2, l3, c3)
    return _tc_final(feats)[:NN, :NN]
